# baseline (speedup 1.0000x reference)
import functools

import jax
import jax.numpy as jnp
from jax import lax
from jax.experimental import pallas as pl
from jax.experimental.pallas import tpu as pltpu
from jax.experimental.pallas import tpu_sc as plsc

_NW = 32
_CH = 128


def _silu(x):
    return x * jax.nn.sigmoid(x)


def _sc_gather(feat, cx, cy, cz, src_pad):
    nw, nchunk, ch = src_pad.shape
    n, d = feat.shape
    epad = nw * nchunk * ch
    deg = 16
    npad = epad // deg
    nrows = ch // deg
    mesh = plsc.VectorSubcoreMesh(core_axis_name="c", subcore_axis_name="s")

    nb = 2
    wrows = nchunk * nrows

    @functools.partial(
        pl.kernel,
        out_type=(
            jax.ShapeDtypeStruct((epad, d), jnp.float32),
            jax.ShapeDtypeStruct((npad, 3 * deg), jnp.float32),
        ),
        mesh=mesh,
        scratch_types=[
            pltpu.VMEM((nchunk, ch), jnp.int32),
            pltpu.VMEM((nb, ch, d), jnp.float32),
            pltpu.VMEM((n,), jnp.float32),
            pltpu.VMEM((n,), jnp.float32),
            pltpu.VMEM((n,), jnp.float32),
            pltpu.VMEM((wrows, 3 * deg), jnp.float32),
            [pltpu.SemaphoreType.DMA] * nb,
            [pltpu.SemaphoreType.DMA] * nb,
        ],
        compiler_params=pltpu.CompilerParams(needs_layout_passes=False),
    )
    def gather_kernel(feat_hbm, cx_hbm, cy_hbm, cz_hbm, src_hbm,
                      gf_hbm, xyz_hbm,
                      idx_v, fbuf, cxv, cyv, czv,
                      xyzacc, gsems, ssems):
        wid = lax.axis_index("s") * 2 + lax.axis_index("c")
        pltpu.sync_copy(src_hbm.at[wid], idx_v)
        base = wid * nchunk
        for b in range(nb):
            pltpu.async_copy(feat_hbm.at[idx_v.at[b]], fbuf.at[b], gsems[b])
        pltpu.sync_copy(cx_hbm, cxv)
        pltpu.sync_copy(cy_hbm, cyv)
        pltpu.sync_copy(cz_hbm, czv)

        def cgather(j, carry):
            for t in range(nrows):
                iv = idx_v[j, pl.ds(t * 16, 16)]
                r = j * nrows + t
                xyzacc[r, pl.ds(0, deg)] = plsc.load_gather(cxv, [iv])
                xyzacc[r, pl.ds(deg, deg)] = plsc.load_gather(cyv, [iv])
                xyzacc[r, pl.ds(2 * deg, deg)] = plsc.load_gather(czv, [iv])
            return carry

        lax.fori_loop(0, nchunk, cgather, 0)
        nrow0 = base * nrows
        pltpu.sync_copy(xyzacc, xyz_hbm.at[pl.ds(nrow0, wrows)])

        def ring(jj, carry):
            j0 = jj * nb
            for b in range(nb):
                j = j0 + b
                pltpu.make_async_copy(
                    feat_hbm.at[idx_v.at[j]], fbuf.at[b], gsems[b]).wait()
                row = (base + j) * ch
                pltpu.async_copy(fbuf.at[b], gf_hbm.at[pl.ds(row, ch)],
                                 ssems[b])
                nxt = j + nb

                @pl.when(nxt < nchunk)
                def _():
                    pltpu.make_async_copy(
                        fbuf.at[b], gf_hbm.at[pl.ds(row, ch)], ssems[b]).wait()
                    pltpu.async_copy(
                        feat_hbm.at[idx_v.at[nxt]], fbuf.at[b], gsems[b])
            return carry

        lax.fori_loop(0, nchunk // nb, ring, 0)
        for b in range(nb):
            pltpu.make_async_copy(
                fbuf.at[b], gf_hbm.at[pl.ds(base * ch, ch)], ssems[b]).wait()

    return gather_kernel(feat, cx, cy, cz, src_pad)


def _tc_project(feat, ew1b, ew1c):
    n, d = feat.shape
    h = ew1b.shape[1]
    bp = 2000
    grid = n // bp

    def kern(feat_ref, wb_ref, wc_ref, zs_ref, zd_ref):
        f = feat_ref[...]
        zs_ref[...] = jnp.dot(f, wb_ref[...], preferred_element_type=jnp.float32)
        zd_ref[...] = jnp.dot(f, wc_ref[...], preferred_element_type=jnp.float32)

    return pl.pallas_call(
        kern,
        grid=(grid,),
        in_specs=[
            pl.BlockSpec((bp, d), lambda i: (i, 0)),
            pl.BlockSpec((d, h), lambda i: (0, 0)),
            pl.BlockSpec((d, h), lambda i: (0, 0)),
        ],
        out_specs=[
            pl.BlockSpec((bp, h), lambda i: (i, 0)),
            pl.BlockSpec((bp, h), lambda i: (i, 0)),
        ],
        out_shape=[
            jax.ShapeDtypeStruct((n, h), jnp.float32),
            jax.ShapeDtypeStruct((n, h), jnp.float32),
        ],
        compiler_params=pltpu.CompilerParams(
            dimension_semantics=("parallel",)),
    )(feat, ew1b, ew1c)


def _tc_total(xyz, deg, n):
    bp = 2000
    grid = n // bp

    def kern(xyz_ref, out_ref):
        @pl.when(pl.program_id(0) == 0)
        def _():
            out_ref[...] = jnp.zeros((1, 1), jnp.float32)

        acc = jnp.float32(0.0)
        for c in range(3):
            x = xyz_ref[:, c * deg:(c + 1) * deg]
            rs = jnp.sum(x, axis=1)
            acc += 2.0 * deg * jnp.sum(x * x) - 2.0 * jnp.sum(rs * rs)
        out_ref[...] += jnp.reshape(acc, (1, 1))

    return pl.pallas_call(
        kern,
        grid=(grid,),
        in_specs=[pl.BlockSpec((bp, 3 * deg), lambda i: (i, 0))],
        out_specs=pl.BlockSpec((1, 1), lambda i: (0, 0)),
        out_shape=jax.ShapeDtypeStruct((1, 1), jnp.float32),
        compiler_params=pltpu.CompilerParams(
            dimension_semantics=("arbitrary",)),
    )(xyz)


def _tc_main(g, feat, coordinate, zdst, xyz, ownm, total, w, bn, deg):
    n, d = feat.shape
    hs = 8
    be = bn * deg
    grid = n // bn

    def kern(g_ref, feat_ref, coord_ref, zdst_ref, xyz_ref,
             ownm_ref, tot_ref,
             r_expand, w1t, b1t, bd2, b2t, ssel,
             bde_sm, bde_mx, bde_mn, bde_sd, esb128, mask8,
             nsw, nsb,
             ew1at, ew1d, eb1, ew2, eb2,
             cw1, cb1, cw2, cb2,
             nw1a, nw1b, nw1c, nb1, nw2, nb2,
             hout_ref, xout_ref):
        inv_total = 1.0 / (tot_ref[0, 0] + 1.0)

        ownm = ownm_ref[...]
        delta = jnp.zeros((be, deg), jnp.float32)
        xis = []
        for c in range(3):
            xc = xyz_ref[:, c * deg:(c + 1) * deg]
            xc_rep = jnp.broadcast_to(
                xc[:, None, :], (bn, deg, deg)).reshape(be, deg)
            xi = jnp.sum(xc_rep * ownm, axis=1, keepdims=True)
            xis.append(xi)
            dcomp = xi - xc_rep
            delta = delta + dcomp * dcomp
        delta = delta * inv_total

        delta_rep = jnp.dot(delta, r_expand[...],
                            preferred_element_type=jnp.float32)
        h1 = _silu(delta_rep * w1t[...] + b1t[...])
        h2 = _silu(jnp.dot(h1, bd2[...],
                           preferred_element_type=jnp.float32) + b2t[...])

        h3 = h2.reshape(bn, deg, deg * hs)
        s1p = jnp.sum(h3, axis=1)
        sq1p = jnp.sum(h3 * h3, axis=1)
        mx1p = jnp.max(h3, axis=1)
        mn1p = jnp.min(h3, axis=1)
        mean1p = s1p * (1.0 / deg)
        std1p = jnp.sqrt(jnp.maximum(
            sq1p * (1.0 / deg) - mean1p * mean1p, 0.0))
        hedp = _silu(
            jnp.dot(s1p, bde_sm[...], preferred_element_type=jnp.float32)
            + jnp.dot(mx1p, bde_mx[...], preferred_element_type=jnp.float32)
            + jnp.dot(mn1p, bde_mn[...], preferred_element_type=jnp.float32)
            + jnp.dot(std1p, bde_sd[...], preferred_element_type=jnp.float32)
            + esb128[...])

        s2 = jnp.dot(hedp, ssel[...], preferred_element_type=jnp.float32)
        sq2 = jnp.dot(hedp * hedp, ssel[...], preferred_element_type=jnp.float32)
        mx2 = hedp
        mn2 = hedp
        width = deg * hs
        while width > hs:
            half = width // 2
            mx2 = jnp.maximum(mx2[:, :half], mx2[:, half:width])
            mn2 = jnp.minimum(mn2[:, :half], mn2[:, half:width])
            width = half
        mean2 = s2 * (1.0 / deg)
        std2 = jnp.sqrt(jnp.maximum(sq2 * (1.0 / deg) - mean2 * mean2, 0.0))
        pna2 = jnp.concatenate([s2, mean2, mx2, mn2, std2], axis=1)
        h_v_dx = _silu(jnp.dot(pna2, nsw[...],
                               preferred_element_type=jnp.float32) + nsb[...])

        cdst = coord_ref[...]
        xi3 = jnp.concatenate(xis, axis=1)
        cdst_rep = jnp.broadcast_to(
            cdst[:, None, :], (bn, deg, 3)).reshape(be, 3)
        dv3 = xi3 - cdst_rep
        sqd = jnp.sum(dv3 * dv3, axis=1, keepdims=True)
        fblk = feat_ref[...]
        zdst = zdst_ref[...]
        zdst_rep = jnp.broadcast_to(
            zdst[:, None, :], (bn, deg, zdst.shape[1])).reshape(be, -1)
        hedp_rep = jnp.broadcast_to(
            hedp[:, None, :], (bn, deg, deg * hs)).reshape(be, deg * hs)
        z1 = (jnp.dot(hedp_rep * mask8[...], ew1at[...],
                      preferred_element_type=jnp.float32)
              + g_ref[...] + zdst_rep + sqd * ew1d[...] + eb1[...])
        h_e = _silu(jnp.dot(_silu(z1), ew2[...],
                            preferred_element_type=jnp.float32) + eb2[...])

        t = _silu(jnp.dot(h_e, cw1[...],
                          preferred_element_type=jnp.float32) + cb1[...])
        coef = jnp.dot(t, cw2[...],
                       preferred_element_type=jnp.float32) + cb2[...]
        x_e = dv3 * coef
        x_agg = jnp.sum(x_e.reshape(bn, deg, 3), axis=1)
        xout_ref[...] = cdst + x_agg

        h_agg = jnp.sum(h_e.reshape(bn, deg, d), axis=1)
        z = (jnp.dot(fblk, nw1a[...], preferred_element_type=jnp.float32)
             + jnp.dot(h_agg, nw1b[...], preferred_element_type=jnp.float32)
             + jnp.dot(h_v_dx, nw1c[...], preferred_element_type=jnp.float32)
             + nb1[...])
        hout_ref[...] = jnp.dot(_silu(z), nw2[...],
                                preferred_element_type=jnp.float32) + nb2[...]

    const = lambda a: pl.BlockSpec(a.shape, lambda i: (0,) * a.ndim)
    weights = [w[k] for k in (
        "r_expand", "w1t", "b1t", "bd2", "b2t", "ssel",
        "bde_sm", "bde_mx", "bde_mn", "bde_sd", "esb128", "mask8",
        "nsw", "nsb",
        "ew1at", "ew1d", "eb1", "ew2", "eb2",
        "cw1", "cb1", "cw2", "cb2",
        "nw1a", "nw1b", "nw1c", "nb1", "nw2", "nb2")]
    in_specs = [
        pl.BlockSpec((be, d), lambda i: (i, 0)),
        pl.BlockSpec((bn, d), lambda i: (i, 0)),
        pl.BlockSpec((bn, 3), lambda i: (i, 0)),
        pl.BlockSpec((bn, d), lambda i: (i, 0)),
        pl.BlockSpec((bn, 3 * deg), lambda i: (i, 0)),
        pl.BlockSpec((be, deg), lambda i: (0, 0)),
        pl.BlockSpec((1, 1), lambda i: (0, 0)),
    ] + [const(a) for a in weights]
    return pl.pallas_call(
        kern,
        grid=(grid,),
        in_specs=in_specs,
        out_specs=[
            pl.BlockSpec((bn, d), lambda i: (i, 0)),
            pl.BlockSpec((bn, 3), lambda i: (i, 0)),
        ],
        out_shape=[
            jax.ShapeDtypeStruct((n, d), jnp.float32),
            jax.ShapeDtypeStruct((n, 3), jnp.float32),
        ],
        compiler_params=pltpu.CompilerParams(
            dimension_semantics=("parallel",)),
    )(g, feat, coordinate, zdst, xyz, ownm, total, *weights)


def _prep_weights(p, d, deg, hs, be):
    h = p["eW2"].shape[0]
    jidx = jnp.arange(deg * hs) // hs
    r_expand = (jnp.arange(deg)[:, None] == jidx[None, :]).astype(jnp.float32)
    ssel = (jnp.arange(deg * hs)[:, None] % hs
            == jnp.arange(hs)[None, :]).astype(jnp.float32)
    bd2 = jnp.kron(jnp.eye(deg, dtype=jnp.float32), p["dW2"])
    eye16 = jnp.eye(deg, dtype=jnp.float32)
    esw = p["esW"]
    mask8 = ((jnp.arange(deg * hs)[None, :] // hs)
             == (jnp.arange(be)[:, None] % deg)).astype(jnp.float32)
    w = {
        "r_expand": r_expand,
        "w1t": jnp.tile(p["dW1"][0], deg)[None, :],
        "b1t": jnp.tile(p["db1"], deg)[None, :],
        "bd2": bd2,
        "b2t": jnp.tile(p["db2"], deg)[None, :],
        "ssel": ssel,
        "bde_sm": jnp.kron(eye16, esw[:hs] + esw[hs:2 * hs] / deg),
        "bde_mx": jnp.kron(eye16, esw[2 * hs:3 * hs]),
        "bde_mn": jnp.kron(eye16, esw[3 * hs:4 * hs]),
        "bde_sd": jnp.kron(eye16, esw[4 * hs:5 * hs]),
        "esb128": jnp.tile(p["esb"], deg)[None, :],
        "mask8": mask8,
        "nsw": p["nsW"],
        "nsb": p["nsb"][None, :],
        "ew1at": jnp.tile(p["eW1"][:hs], (deg, 1)),
        "ew1d": p["eW1"][hs + 2 * d:hs + 2 * d + 1],
        "eb1": p["eb1"][None, :],
        "ew2": p["eW2"],
        "eb2": p["eb2"][None, :],
        "cw1": p["cW1"],
        "cb1": p["cb1"][None, :],
        "cw2": p["cW2"],
        "cb2": p["cb2"][None, :],
        "nw1a": p["nW1"][:d],
        "nw1b": p["nW1"][d:2 * d],
        "nw1c": p["nW1"][2 * d:2 * d + hs],
        "nb1": p["nb1"][None, :],
        "nw2": p["nW2"],
        "nb2": p["nb2"][None, :],
    }
    return w


def kernel(feat, coordinate, edge_index, params):
    n, d = feat.shape
    e = edge_index.shape[1]
    deg = e // n
    hs = params["dW2"].shape[0]
    src = edge_index[0].astype(jnp.int32)

    nchunk = -(-e // (_NW * _CH))
    epad = _NW * _CH * nchunk
    src_pad = jnp.pad(src, (0, epad - e)).reshape(_NW, nchunk, _CH)

    zsrc, zdst = _tc_project(
        feat, params["eW1"][hs:hs + d], params["eW1"][hs + d:hs + 2 * d])
    g, xyz = _sc_gather(
        zsrc, coordinate[:, 0], coordinate[:, 1], coordinate[:, 2], src_pad)

    total = _tc_total(xyz, deg, n)
    bn = 200
    w = _prep_weights(params, d, deg, hs, bn * deg)
    ownm = (jnp.arange(bn * deg)[:, None] % deg
            == jnp.arange(deg)[None, :]).astype(jnp.float32)
    h_new, x_new = _tc_main(
        g, feat, coordinate, zdst, xyz, ownm, total, w, bn, deg)
    return h_new, x_new


# speedup vs baseline: 1.0198x; 1.0198x over previous
import functools

import jax
import jax.numpy as jnp
from jax import lax
from jax.experimental import pallas as pl
from jax.experimental.pallas import tpu as pltpu
from jax.experimental.pallas import tpu_sc as plsc

_NW = 32
_CH = 128


def _silu(x):
    return x * jax.nn.sigmoid(x)


def _sc_gather(feat, cx, cy, cz, src_pad):
    nw, nchunk, ch = src_pad.shape
    n, d = feat.shape
    epad = nw * nchunk * ch
    deg = 16
    npad = epad // deg
    nrows = ch // deg
    mesh = plsc.VectorSubcoreMesh(core_axis_name="c", subcore_axis_name="s")

    nb = 2
    wrows = nchunk * nrows

    @functools.partial(
        pl.kernel,
        out_type=(
            jax.ShapeDtypeStruct((epad, d), jnp.float32),
            jax.ShapeDtypeStruct((npad, 3 * deg), jnp.float32),
        ),
        mesh=mesh,
        scratch_types=[
            pltpu.VMEM((nchunk, ch), jnp.int32),
            pltpu.VMEM((nb, ch, d), jnp.float32),
            pltpu.VMEM((n,), jnp.float32),
            pltpu.VMEM((n,), jnp.float32),
            pltpu.VMEM((n,), jnp.float32),
            pltpu.VMEM((wrows, 3 * deg), jnp.float32),
            [pltpu.SemaphoreType.DMA] * nb,
            [pltpu.SemaphoreType.DMA] * nb,
        ],
        compiler_params=pltpu.CompilerParams(needs_layout_passes=False),
    )
    def gather_kernel(feat_hbm, cx_hbm, cy_hbm, cz_hbm, src_hbm,
                      gf_hbm, xyz_hbm,
                      idx_v, fbuf, cxv, cyv, czv,
                      xyzacc, gsems, ssems):
        wid = lax.axis_index("s") * 2 + lax.axis_index("c")
        pltpu.sync_copy(src_hbm.at[wid], idx_v)
        base = wid * nchunk
        for b in range(nb):
            pltpu.async_copy(feat_hbm.at[idx_v.at[b]], fbuf.at[b], gsems[b])
        pltpu.sync_copy(cx_hbm, cxv)
        pltpu.sync_copy(cy_hbm, cyv)
        pltpu.sync_copy(cz_hbm, czv)

        def ring(jj, carry):
            j0 = jj * nb
            for b in range(nb):
                j = j0 + b
                pltpu.make_async_copy(
                    feat_hbm.at[idx_v.at[j]], fbuf.at[b], gsems[b]).wait()
                row = (base + j) * ch
                pltpu.async_copy(fbuf.at[b], gf_hbm.at[pl.ds(row, ch)],
                                 ssems[b])
                for t in range(nrows):
                    iv = idx_v[j, pl.ds(t * 16, 16)]
                    r = j * nrows + t
                    xyzacc[r, pl.ds(0, deg)] = plsc.load_gather(cxv, [iv])
                    xyzacc[r, pl.ds(deg, deg)] = plsc.load_gather(cyv, [iv])
                    xyzacc[r, pl.ds(2 * deg, deg)] = plsc.load_gather(czv, [iv])
                nxt = j + nb

                @pl.when(nxt < nchunk)
                def _():
                    pltpu.make_async_copy(
                        fbuf.at[b], gf_hbm.at[pl.ds(row, ch)], ssems[b]).wait()
                    pltpu.async_copy(
                        feat_hbm.at[idx_v.at[nxt]], fbuf.at[b], gsems[b])
            return carry

        lax.fori_loop(0, nchunk // nb, ring, 0)
        for b in range(nb):
            pltpu.make_async_copy(
                fbuf.at[b], gf_hbm.at[pl.ds(base * ch, ch)], ssems[b]).wait()
        nrow0 = base * nrows
        pltpu.sync_copy(xyzacc, xyz_hbm.at[pl.ds(nrow0, wrows)])

    return gather_kernel(feat, cx, cy, cz, src_pad)


def _tc_project(feat, ew1b, ew1c):
    n, d = feat.shape
    h = ew1b.shape[1]
    bp = 2000
    grid = n // bp

    def kern(feat_ref, wb_ref, wc_ref, zs_ref, zd_ref):
        f = feat_ref[...]
        zs_ref[...] = jnp.dot(f, wb_ref[...], preferred_element_type=jnp.float32)
        zd_ref[...] = jnp.dot(f, wc_ref[...], preferred_element_type=jnp.float32)

    return pl.pallas_call(
        kern,
        grid=(grid,),
        in_specs=[
            pl.BlockSpec((bp, d), lambda i: (i, 0)),
            pl.BlockSpec((d, h), lambda i: (0, 0)),
            pl.BlockSpec((d, h), lambda i: (0, 0)),
        ],
        out_specs=[
            pl.BlockSpec((bp, h), lambda i: (i, 0)),
            pl.BlockSpec((bp, h), lambda i: (i, 0)),
        ],
        out_shape=[
            jax.ShapeDtypeStruct((n, h), jnp.float32),
            jax.ShapeDtypeStruct((n, h), jnp.float32),
        ],
        compiler_params=pltpu.CompilerParams(
            dimension_semantics=("parallel",)),
    )(feat, ew1b, ew1c)


def _tc_total(xyz, deg, n):
    bp = 2000
    grid = n // bp

    def kern(xyz_ref, out_ref):
        @pl.when(pl.program_id(0) == 0)
        def _():
            out_ref[...] = jnp.zeros((1, 1), jnp.float32)

        acc = jnp.float32(0.0)
        for c in range(3):
            x = xyz_ref[:, c * deg:(c + 1) * deg]
            rs = jnp.sum(x, axis=1)
            acc += 2.0 * deg * jnp.sum(x * x) - 2.0 * jnp.sum(rs * rs)
        out_ref[...] += jnp.reshape(acc, (1, 1))

    return pl.pallas_call(
        kern,
        grid=(grid,),
        in_specs=[pl.BlockSpec((bp, 3 * deg), lambda i: (i, 0))],
        out_specs=pl.BlockSpec((1, 1), lambda i: (0, 0)),
        out_shape=jax.ShapeDtypeStruct((1, 1), jnp.float32),
        compiler_params=pltpu.CompilerParams(
            dimension_semantics=("arbitrary",)),
    )(xyz)


def _tc_main(g, feat, coordinate, zdst, xyz, ownm, total, w, bn, deg):
    n, d = feat.shape
    hs = 8
    be = bn * deg
    grid = n // bn

    def kern(g_ref, feat_ref, coord_ref, zdst_ref, xyz_ref,
             ownm_ref, tot_ref,
             r_expand, w1t, b1t, bd2, b2t, ssel,
             bde_sm, bde_mx, bde_mn, bde_sd, esb128, mask8,
             nsw, nsb,
             ew1at, ew1d, eb1, ew2, eb2,
             cw1, cb1, cw2, cb2,
             nw1a, nw1b, nw1c, nb1, nw2, nb2,
             hout_ref, xout_ref):
        inv_total = 1.0 / (tot_ref[0, 0] + 1.0)

        ownm = ownm_ref[...]
        delta = jnp.zeros((be, deg), jnp.float32)
        xis = []
        for c in range(3):
            xc = xyz_ref[:, c * deg:(c + 1) * deg]
            xc_rep = jnp.broadcast_to(
                xc[:, None, :], (bn, deg, deg)).reshape(be, deg)
            xi = jnp.sum(xc_rep * ownm, axis=1, keepdims=True)
            xis.append(xi)
            dcomp = xi - xc_rep
            delta = delta + dcomp * dcomp
        delta = delta * inv_total

        delta_rep = jnp.dot(delta, r_expand[...],
                            preferred_element_type=jnp.float32)
        h1 = _silu(delta_rep * w1t[...] + b1t[...])
        h2 = _silu(jnp.dot(h1, bd2[...],
                           preferred_element_type=jnp.float32) + b2t[...])

        h3 = h2.reshape(bn, deg, deg * hs)
        s1p = jnp.sum(h3, axis=1)
        sq1p = jnp.sum(h3 * h3, axis=1)
        mx1p = jnp.max(h3, axis=1)
        mn1p = jnp.min(h3, axis=1)
        mean1p = s1p * (1.0 / deg)
        std1p = jnp.sqrt(jnp.maximum(
            sq1p * (1.0 / deg) - mean1p * mean1p, 0.0))
        hedp = _silu(
            jnp.dot(s1p, bde_sm[...], preferred_element_type=jnp.float32)
            + jnp.dot(mx1p, bde_mx[...], preferred_element_type=jnp.float32)
            + jnp.dot(mn1p, bde_mn[...], preferred_element_type=jnp.float32)
            + jnp.dot(std1p, bde_sd[...], preferred_element_type=jnp.float32)
            + esb128[...])

        s2 = jnp.dot(hedp, ssel[...], preferred_element_type=jnp.float32)
        sq2 = jnp.dot(hedp * hedp, ssel[...], preferred_element_type=jnp.float32)
        mx2 = hedp
        mn2 = hedp
        width = deg * hs
        while width > hs:
            half = width // 2
            mx2 = jnp.maximum(mx2[:, :half], mx2[:, half:width])
            mn2 = jnp.minimum(mn2[:, :half], mn2[:, half:width])
            width = half
        mean2 = s2 * (1.0 / deg)
        std2 = jnp.sqrt(jnp.maximum(sq2 * (1.0 / deg) - mean2 * mean2, 0.0))
        pna2 = jnp.concatenate([s2, mean2, mx2, mn2, std2], axis=1)
        h_v_dx = _silu(jnp.dot(pna2, nsw[...],
                               preferred_element_type=jnp.float32) + nsb[...])

        cdst = coord_ref[...]
        xi3 = jnp.concatenate(xis, axis=1)
        cdst_rep = jnp.broadcast_to(
            cdst[:, None, :], (bn, deg, 3)).reshape(be, 3)
        dv3 = xi3 - cdst_rep
        sqd = jnp.sum(dv3 * dv3, axis=1, keepdims=True)
        fblk = feat_ref[...]
        zdst = zdst_ref[...]
        zdst_rep = jnp.broadcast_to(
            zdst[:, None, :], (bn, deg, zdst.shape[1])).reshape(be, -1)
        hedp_rep = jnp.broadcast_to(
            hedp[:, None, :], (bn, deg, deg * hs)).reshape(be, deg * hs)
        z1 = (jnp.dot(hedp_rep * mask8[...], ew1at[...],
                      preferred_element_type=jnp.float32)
              + g_ref[...] + zdst_rep + sqd * ew1d[...] + eb1[...])
        h_e = _silu(jnp.dot(_silu(z1), ew2[...],
                            preferred_element_type=jnp.float32) + eb2[...])

        t = _silu(jnp.dot(h_e, cw1[...],
                          preferred_element_type=jnp.float32) + cb1[...])
        coef = jnp.dot(t, cw2[...],
                       preferred_element_type=jnp.float32) + cb2[...]
        x_e = dv3 * coef
        x_agg = jnp.sum(x_e.reshape(bn, deg, 3), axis=1)
        xout_ref[...] = cdst + x_agg

        h_agg = jnp.sum(h_e.reshape(bn, deg, d), axis=1)
        z = (jnp.dot(fblk, nw1a[...], preferred_element_type=jnp.float32)
             + jnp.dot(h_agg, nw1b[...], preferred_element_type=jnp.float32)
             + jnp.dot(h_v_dx, nw1c[...], preferred_element_type=jnp.float32)
             + nb1[...])
        hout_ref[...] = jnp.dot(_silu(z), nw2[...],
                                preferred_element_type=jnp.float32) + nb2[...]

    const = lambda a: pl.BlockSpec(a.shape, lambda i: (0,) * a.ndim)
    weights = [w[k] for k in (
        "r_expand", "w1t", "b1t", "bd2", "b2t", "ssel",
        "bde_sm", "bde_mx", "bde_mn", "bde_sd", "esb128", "mask8",
        "nsw", "nsb",
        "ew1at", "ew1d", "eb1", "ew2", "eb2",
        "cw1", "cb1", "cw2", "cb2",
        "nw1a", "nw1b", "nw1c", "nb1", "nw2", "nb2")]
    in_specs = [
        pl.BlockSpec((be, d), lambda i: (i, 0)),
        pl.BlockSpec((bn, d), lambda i: (i, 0)),
        pl.BlockSpec((bn, 3), lambda i: (i, 0)),
        pl.BlockSpec((bn, d), lambda i: (i, 0)),
        pl.BlockSpec((bn, 3 * deg), lambda i: (i, 0)),
        pl.BlockSpec((be, deg), lambda i: (0, 0)),
        pl.BlockSpec((1, 1), lambda i: (0, 0)),
    ] + [const(a) for a in weights]
    return pl.pallas_call(
        kern,
        grid=(grid,),
        in_specs=in_specs,
        out_specs=[
            pl.BlockSpec((bn, d), lambda i: (i, 0)),
            pl.BlockSpec((bn, 3), lambda i: (i, 0)),
        ],
        out_shape=[
            jax.ShapeDtypeStruct((n, d), jnp.float32),
            jax.ShapeDtypeStruct((n, 3), jnp.float32),
        ],
        compiler_params=pltpu.CompilerParams(
            dimension_semantics=("parallel",)),
    )(g, feat, coordinate, zdst, xyz, ownm, total, *weights)


def _prep_weights(p, d, deg, hs, be):
    h = p["eW2"].shape[0]
    jidx = jnp.arange(deg * hs) // hs
    r_expand = (jnp.arange(deg)[:, None] == jidx[None, :]).astype(jnp.float32)
    ssel = (jnp.arange(deg * hs)[:, None] % hs
            == jnp.arange(hs)[None, :]).astype(jnp.float32)
    bd2 = jnp.kron(jnp.eye(deg, dtype=jnp.float32), p["dW2"])
    eye16 = jnp.eye(deg, dtype=jnp.float32)
    esw = p["esW"]
    mask8 = ((jnp.arange(deg * hs)[None, :] // hs)
             == (jnp.arange(be)[:, None] % deg)).astype(jnp.float32)
    w = {
        "r_expand": r_expand,
        "w1t": jnp.tile(p["dW1"][0], deg)[None, :],
        "b1t": jnp.tile(p["db1"], deg)[None, :],
        "bd2": bd2,
        "b2t": jnp.tile(p["db2"], deg)[None, :],
        "ssel": ssel,
        "bde_sm": jnp.kron(eye16, esw[:hs] + esw[hs:2 * hs] / deg),
        "bde_mx": jnp.kron(eye16, esw[2 * hs:3 * hs]),
        "bde_mn": jnp.kron(eye16, esw[3 * hs:4 * hs]),
        "bde_sd": jnp.kron(eye16, esw[4 * hs:5 * hs]),
        "esb128": jnp.tile(p["esb"], deg)[None, :],
        "mask8": mask8,
        "nsw": p["nsW"],
        "nsb": p["nsb"][None, :],
        "ew1at": jnp.tile(p["eW1"][:hs], (deg, 1)),
        "ew1d": p["eW1"][hs + 2 * d:hs + 2 * d + 1],
        "eb1": p["eb1"][None, :],
        "ew2": p["eW2"],
        "eb2": p["eb2"][None, :],
        "cw1": p["cW1"],
        "cb1": p["cb1"][None, :],
        "cw2": p["cW2"],
        "cb2": p["cb2"][None, :],
        "nw1a": p["nW1"][:d],
        "nw1b": p["nW1"][d:2 * d],
        "nw1c": p["nW1"][2 * d:2 * d + hs],
        "nb1": p["nb1"][None, :],
        "nw2": p["nW2"],
        "nb2": p["nb2"][None, :],
    }
    return w


def kernel(feat, coordinate, edge_index, params):
    n, d = feat.shape
    e = edge_index.shape[1]
    deg = e // n
    hs = params["dW2"].shape[0]
    src = edge_index[0].astype(jnp.int32)

    nchunk = -(-e // (_NW * _CH))
    epad = _NW * _CH * nchunk
    src_pad = jnp.pad(src, (0, epad - e)).reshape(_NW, nchunk, _CH)

    zsrc, zdst = _tc_project(
        feat, params["eW1"][hs:hs + d], params["eW1"][hs + d:hs + 2 * d])
    g, xyz = _sc_gather(
        zsrc, coordinate[:, 0], coordinate[:, 1], coordinate[:, 2], src_pad)

    total = _tc_total(xyz, deg, n)
    bn = 200
    w = _prep_weights(params, d, deg, hs, bn * deg)
    ownm = (jnp.arange(bn * deg)[:, None] % deg
            == jnp.arange(deg)[None, :]).astype(jnp.float32)
    h_new, x_new = _tc_main(
        g, feat, coordinate, zdst, xyz, ownm, total, w, bn, deg)
    return h_new, x_new


# speedup vs baseline: 1.0237x; 1.0038x over previous
import functools

import jax
import jax.numpy as jnp
from jax import lax
from jax.experimental import pallas as pl
from jax.experimental.pallas import tpu as pltpu
from jax.experimental.pallas import tpu_sc as plsc

_NW = 32
_CH = 128


def _silu(x):
    return x * jax.nn.sigmoid(x)


def _sc_gather(feat, cx, cy, cz, src_pad):
    nw, nchunk, ch = src_pad.shape
    n, d = feat.shape
    epad = nw * nchunk * ch
    deg = 16
    npad = epad // deg
    nrows = ch // deg
    mesh = plsc.VectorSubcoreMesh(core_axis_name="c", subcore_axis_name="s")

    nb = 2
    wrows = nchunk * nrows

    @functools.partial(
        pl.kernel,
        out_type=(
            jax.ShapeDtypeStruct((epad, d), jnp.float32),
            jax.ShapeDtypeStruct((npad, 3 * deg), jnp.float32),
        ),
        mesh=mesh,
        scratch_types=[
            pltpu.VMEM((nchunk, ch), jnp.int32),
            pltpu.VMEM((nb, ch, d), jnp.float32),
            pltpu.VMEM((n,), jnp.float32),
            pltpu.VMEM((n,), jnp.float32),
            pltpu.VMEM((n,), jnp.float32),
            pltpu.VMEM((nb, nrows, 3 * deg), jnp.float32),
            [pltpu.SemaphoreType.DMA] * nb,
            [pltpu.SemaphoreType.DMA] * nb,
            [pltpu.SemaphoreType.DMA] * nb,
        ],
        compiler_params=pltpu.CompilerParams(needs_layout_passes=False),
    )
    def gather_kernel(feat_hbm, cx_hbm, cy_hbm, cz_hbm, src_hbm,
                      gf_hbm, xyz_hbm,
                      idx_v, fbuf, cxv, cyv, czv,
                      cbuf, gsems, ssems, csems):
        wid = lax.axis_index("s") * 2 + lax.axis_index("c")
        pltpu.sync_copy(src_hbm.at[wid], idx_v)
        base = wid * nchunk
        for b in range(nb):
            pltpu.async_copy(feat_hbm.at[idx_v.at[b]], fbuf.at[b], gsems[b])
        pltpu.sync_copy(cx_hbm, cxv)
        pltpu.sync_copy(cy_hbm, cyv)
        pltpu.sync_copy(cz_hbm, czv)

        def ring(jj, carry):
            j0 = jj * nb
            for b in range(nb):
                j = j0 + b
                pltpu.make_async_copy(
                    feat_hbm.at[idx_v.at[j]], fbuf.at[b], gsems[b]).wait()
                row = (base + j) * ch
                pltpu.async_copy(fbuf.at[b], gf_hbm.at[pl.ds(row, ch)],
                                 ssems[b])
                cb = cbuf.at[b]
                nrow = (base + j) * nrows

                @pl.when(j >= nb)
                def _():
                    pltpu.make_async_copy(
                        cb, xyz_hbm.at[pl.ds(nrow, nrows)], csems[b]).wait()

                for t in range(nrows):
                    iv = idx_v[j, pl.ds(t * 16, 16)]
                    cb[t, pl.ds(0, deg)] = plsc.load_gather(cxv, [iv])
                    cb[t, pl.ds(deg, deg)] = plsc.load_gather(cyv, [iv])
                    cb[t, pl.ds(2 * deg, deg)] = plsc.load_gather(czv, [iv])
                pltpu.async_copy(cb, xyz_hbm.at[pl.ds(nrow, nrows)], csems[b])
                nxt = j + nb

                @pl.when(nxt < nchunk)
                def _():
                    pltpu.make_async_copy(
                        fbuf.at[b], gf_hbm.at[pl.ds(row, ch)], ssems[b]).wait()
                    pltpu.async_copy(
                        feat_hbm.at[idx_v.at[nxt]], fbuf.at[b], gsems[b])
            return carry

        lax.fori_loop(0, nchunk // nb, ring, 0)
        for b in range(nb):
            pltpu.make_async_copy(
                fbuf.at[b], gf_hbm.at[pl.ds(base * ch, ch)], ssems[b]).wait()
            pltpu.make_async_copy(
                cbuf.at[b], xyz_hbm.at[pl.ds(base * nrows, nrows)],
                csems[b]).wait()

    return gather_kernel(feat, cx, cy, cz, src_pad)


def _tc_project(feat, ew1b, ew1c):
    n, d = feat.shape
    h = ew1b.shape[1]
    bp = 2000
    grid = n // bp

    def kern(feat_ref, wb_ref, wc_ref, zs_ref, zd_ref):
        f = feat_ref[...]
        zs_ref[...] = jnp.dot(f, wb_ref[...], preferred_element_type=jnp.float32)
        zd_ref[...] = jnp.dot(f, wc_ref[...], preferred_element_type=jnp.float32)

    return pl.pallas_call(
        kern,
        grid=(grid,),
        in_specs=[
            pl.BlockSpec((bp, d), lambda i: (i, 0)),
            pl.BlockSpec((d, h), lambda i: (0, 0)),
            pl.BlockSpec((d, h), lambda i: (0, 0)),
        ],
        out_specs=[
            pl.BlockSpec((bp, h), lambda i: (i, 0)),
            pl.BlockSpec((bp, h), lambda i: (i, 0)),
        ],
        out_shape=[
            jax.ShapeDtypeStruct((n, h), jnp.float32),
            jax.ShapeDtypeStruct((n, h), jnp.float32),
        ],
        compiler_params=pltpu.CompilerParams(
            dimension_semantics=("parallel",)),
    )(feat, ew1b, ew1c)


def _tc_total(xyz, deg, n):
    bp = 2000
    grid = n // bp

    def kern(xyz_ref, out_ref):
        @pl.when(pl.program_id(0) == 0)
        def _():
            out_ref[...] = jnp.zeros((1, 1), jnp.float32)

        acc = jnp.float32(0.0)
        for c in range(3):
            x = xyz_ref[:, c * deg:(c + 1) * deg]
            rs = jnp.sum(x, axis=1)
            acc += 2.0 * deg * jnp.sum(x * x) - 2.0 * jnp.sum(rs * rs)
        out_ref[...] += jnp.reshape(acc, (1, 1))

    return pl.pallas_call(
        kern,
        grid=(grid,),
        in_specs=[pl.BlockSpec((bp, 3 * deg), lambda i: (i, 0))],
        out_specs=pl.BlockSpec((1, 1), lambda i: (0, 0)),
        out_shape=jax.ShapeDtypeStruct((1, 1), jnp.float32),
        compiler_params=pltpu.CompilerParams(
            dimension_semantics=("arbitrary",)),
    )(xyz)


def _tc_main(g, feat, coordinate, zdst, xyz, ownm, total, w, bn, deg):
    n, d = feat.shape
    hs = 8
    be = bn * deg
    grid = n // bn

    def kern(g_ref, feat_ref, coord_ref, zdst_ref, xyz_ref,
             ownm_ref, tot_ref,
             r_expand, w1t, b1t, bd2, b2t, ssel,
             bde_sm, bde_mx, bde_mn, bde_sd, esb128, mask8,
             nsw, nsb,
             ew1at, ew1d, eb1, ew2, eb2,
             cw1, cb1, cw2, cb2,
             nw1a, nw1b, nw1c, nb1, nw2, nb2,
             hout_ref, xout_ref):
        inv_total = 1.0 / (tot_ref[0, 0] + 1.0)

        ownm = ownm_ref[...]
        delta = jnp.zeros((be, deg), jnp.float32)
        xis = []
        for c in range(3):
            xc = xyz_ref[:, c * deg:(c + 1) * deg]
            xc_rep = jnp.broadcast_to(
                xc[:, None, :], (bn, deg, deg)).reshape(be, deg)
            xi = jnp.sum(xc_rep * ownm, axis=1, keepdims=True)
            xis.append(xi)
            dcomp = xi - xc_rep
            delta = delta + dcomp * dcomp
        delta = delta * inv_total

        delta_rep = jnp.dot(delta, r_expand[...],
                            preferred_element_type=jnp.float32)
        h1 = _silu(delta_rep * w1t[...] + b1t[...])
        h2 = _silu(jnp.dot(h1, bd2[...],
                           preferred_element_type=jnp.float32) + b2t[...])

        h3 = h2.reshape(bn, deg, deg * hs)
        s1p = jnp.sum(h3, axis=1)
        sq1p = jnp.sum(h3 * h3, axis=1)
        mx1p = jnp.max(h3, axis=1)
        mn1p = jnp.min(h3, axis=1)
        mean1p = s1p * (1.0 / deg)
        std1p = jnp.sqrt(jnp.maximum(
            sq1p * (1.0 / deg) - mean1p * mean1p, 0.0))
        hedp = _silu(
            jnp.dot(s1p, bde_sm[...], preferred_element_type=jnp.float32)
            + jnp.dot(mx1p, bde_mx[...], preferred_element_type=jnp.float32)
            + jnp.dot(mn1p, bde_mn[...], preferred_element_type=jnp.float32)
            + jnp.dot(std1p, bde_sd[...], preferred_element_type=jnp.float32)
            + esb128[...])

        s2 = jnp.dot(hedp, ssel[...], preferred_element_type=jnp.float32)
        sq2 = jnp.dot(hedp * hedp, ssel[...], preferred_element_type=jnp.float32)
        mx2 = hedp
        mn2 = hedp
        width = deg * hs
        while width > hs:
            half = width // 2
            mx2 = jnp.maximum(mx2[:, :half], mx2[:, half:width])
            mn2 = jnp.minimum(mn2[:, :half], mn2[:, half:width])
            width = half
        mean2 = s2 * (1.0 / deg)
        std2 = jnp.sqrt(jnp.maximum(sq2 * (1.0 / deg) - mean2 * mean2, 0.0))
        pna2 = jnp.concatenate([s2, mean2, mx2, mn2, std2], axis=1)
        h_v_dx = _silu(jnp.dot(pna2, nsw[...],
                               preferred_element_type=jnp.float32) + nsb[...])

        cdst = coord_ref[...]
        xi3 = jnp.concatenate(xis, axis=1)
        cdst_rep = jnp.broadcast_to(
            cdst[:, None, :], (bn, deg, 3)).reshape(be, 3)
        dv3 = xi3 - cdst_rep
        sqd = jnp.sum(dv3 * dv3, axis=1, keepdims=True)
        fblk = feat_ref[...]
        zdst = zdst_ref[...]
        zdst_rep = jnp.broadcast_to(
            zdst[:, None, :], (bn, deg, zdst.shape[1])).reshape(be, -1)
        hedp_rep = jnp.broadcast_to(
            hedp[:, None, :], (bn, deg, deg * hs)).reshape(be, deg * hs)
        z1 = (jnp.dot(hedp_rep * mask8[...], ew1at[...],
                      preferred_element_type=jnp.float32)
              + g_ref[...] + zdst_rep + sqd * ew1d[...] + eb1[...])
        h_e = _silu(jnp.dot(_silu(z1), ew2[...],
                            preferred_element_type=jnp.float32) + eb2[...])

        t = _silu(jnp.dot(h_e, cw1[...],
                          preferred_element_type=jnp.float32) + cb1[...])
        coef = jnp.dot(t, cw2[...],
                       preferred_element_type=jnp.float32) + cb2[...]
        x_e = dv3 * coef
        x_agg = jnp.sum(x_e.reshape(bn, deg, 3), axis=1)
        xout_ref[...] = cdst + x_agg

        h_agg = jnp.sum(h_e.reshape(bn, deg, d), axis=1)
        z = (jnp.dot(fblk, nw1a[...], preferred_element_type=jnp.float32)
             + jnp.dot(h_agg, nw1b[...], preferred_element_type=jnp.float32)
             + jnp.dot(h_v_dx, nw1c[...], preferred_element_type=jnp.float32)
             + nb1[...])
        hout_ref[...] = jnp.dot(_silu(z), nw2[...],
                                preferred_element_type=jnp.float32) + nb2[...]

    const = lambda a: pl.BlockSpec(a.shape, lambda i: (0,) * a.ndim)
    weights = [w[k] for k in (
        "r_expand", "w1t", "b1t", "bd2", "b2t", "ssel",
        "bde_sm", "bde_mx", "bde_mn", "bde_sd", "esb128", "mask8",
        "nsw", "nsb",
        "ew1at", "ew1d", "eb1", "ew2", "eb2",
        "cw1", "cb1", "cw2", "cb2",
        "nw1a", "nw1b", "nw1c", "nb1", "nw2", "nb2")]
    in_specs = [
        pl.BlockSpec((be, d), lambda i: (i, 0)),
        pl.BlockSpec((bn, d), lambda i: (i, 0)),
        pl.BlockSpec((bn, 3), lambda i: (i, 0)),
        pl.BlockSpec((bn, d), lambda i: (i, 0)),
        pl.BlockSpec((bn, 3 * deg), lambda i: (i, 0)),
        pl.BlockSpec((be, deg), lambda i: (0, 0)),
        pl.BlockSpec((1, 1), lambda i: (0, 0)),
    ] + [const(a) for a in weights]
    return pl.pallas_call(
        kern,
        grid=(grid,),
        in_specs=in_specs,
        out_specs=[
            pl.BlockSpec((bn, d), lambda i: (i, 0)),
            pl.BlockSpec((bn, 3), lambda i: (i, 0)),
        ],
        out_shape=[
            jax.ShapeDtypeStruct((n, d), jnp.float32),
            jax.ShapeDtypeStruct((n, 3), jnp.float32),
        ],
        compiler_params=pltpu.CompilerParams(
            dimension_semantics=("parallel",)),
    )(g, feat, coordinate, zdst, xyz, ownm, total, *weights)


def _prep_weights(p, d, deg, hs, be):
    h = p["eW2"].shape[0]
    jidx = jnp.arange(deg * hs) // hs
    r_expand = (jnp.arange(deg)[:, None] == jidx[None, :]).astype(jnp.float32)
    ssel = (jnp.arange(deg * hs)[:, None] % hs
            == jnp.arange(hs)[None, :]).astype(jnp.float32)
    bd2 = jnp.kron(jnp.eye(deg, dtype=jnp.float32), p["dW2"])
    eye16 = jnp.eye(deg, dtype=jnp.float32)
    esw = p["esW"]
    mask8 = ((jnp.arange(deg * hs)[None, :] // hs)
             == (jnp.arange(be)[:, None] % deg)).astype(jnp.float32)
    w = {
        "r_expand": r_expand,
        "w1t": jnp.tile(p["dW1"][0], deg)[None, :],
        "b1t": jnp.tile(p["db1"], deg)[None, :],
        "bd2": bd2,
        "b2t": jnp.tile(p["db2"], deg)[None, :],
        "ssel": ssel,
        "bde_sm": jnp.kron(eye16, esw[:hs] + esw[hs:2 * hs] / deg),
        "bde_mx": jnp.kron(eye16, esw[2 * hs:3 * hs]),
        "bde_mn": jnp.kron(eye16, esw[3 * hs:4 * hs]),
        "bde_sd": jnp.kron(eye16, esw[4 * hs:5 * hs]),
        "esb128": jnp.tile(p["esb"], deg)[None, :],
        "mask8": mask8,
        "nsw": p["nsW"],
        "nsb": p["nsb"][None, :],
        "ew1at": jnp.tile(p["eW1"][:hs], (deg, 1)),
        "ew1d": p["eW1"][hs + 2 * d:hs + 2 * d + 1],
        "eb1": p["eb1"][None, :],
        "ew2": p["eW2"],
        "eb2": p["eb2"][None, :],
        "cw1": p["cW1"],
        "cb1": p["cb1"][None, :],
        "cw2": p["cW2"],
        "cb2": p["cb2"][None, :],
        "nw1a": p["nW1"][:d],
        "nw1b": p["nW1"][d:2 * d],
        "nw1c": p["nW1"][2 * d:2 * d + hs],
        "nb1": p["nb1"][None, :],
        "nw2": p["nW2"],
        "nb2": p["nb2"][None, :],
    }
    return w


def kernel(feat, coordinate, edge_index, params):
    n, d = feat.shape
    e = edge_index.shape[1]
    deg = e // n
    hs = params["dW2"].shape[0]
    src = edge_index[0].astype(jnp.int32)

    nchunk = -(-e // (_NW * _CH))
    epad = _NW * _CH * nchunk
    src_pad = jnp.pad(src, (0, epad - e)).reshape(_NW, nchunk, _CH)

    zsrc, zdst = _tc_project(
        feat, params["eW1"][hs:hs + d], params["eW1"][hs + d:hs + 2 * d])
    g, xyz = _sc_gather(
        zsrc, coordinate[:, 0], coordinate[:, 1], coordinate[:, 2], src_pad)

    total = _tc_total(xyz, deg, n)
    bn = 200
    w = _prep_weights(params, d, deg, hs, bn * deg)
    ownm = (jnp.arange(bn * deg)[:, None] % deg
            == jnp.arange(deg)[None, :]).astype(jnp.float32)
    h_new, x_new = _tc_main(
        g, feat, coordinate, zdst, xyz, ownm, total, w, bn, deg)
    return h_new, x_new


# speedup vs baseline: 1.2616x; 1.2325x over previous
import functools

import jax
import jax.numpy as jnp
from jax import lax
from jax.experimental import pallas as pl
from jax.experimental.pallas import tpu as pltpu
from jax.experimental.pallas import tpu_sc as plsc

_NW = 32
_CH = 128


def _silu(x):
    return x * jax.nn.sigmoid(x)


def _sc_gather(feat, cx, cy, cz, src_pad):
    nw, nchunk, ch = src_pad.shape
    n, d = feat.shape
    epad = nw * nchunk * ch
    deg = 16
    npad = epad // deg
    nrows = ch // deg
    mesh = plsc.VectorSubcoreMesh(core_axis_name="c", subcore_axis_name="s")

    nb = 2
    wrows = nchunk * nrows

    @functools.partial(
        pl.kernel,
        out_type=(
            jax.ShapeDtypeStruct((epad, d), jnp.float32),
            jax.ShapeDtypeStruct((npad, deg), jnp.float32),
            jax.ShapeDtypeStruct((npad, deg), jnp.float32),
            jax.ShapeDtypeStruct((npad, deg), jnp.float32),
        ),
        mesh=mesh,
        scratch_types=[
            pltpu.VMEM((nchunk, ch), jnp.int32),
            pltpu.VMEM((nb, ch, d), jnp.float32),
            pltpu.VMEM((n,), jnp.float32),
            pltpu.VMEM((n,), jnp.float32),
            pltpu.VMEM((n,), jnp.float32),
            pltpu.VMEM((nb, nrows, deg), jnp.float32),
            pltpu.VMEM((nb, nrows, deg), jnp.float32),
            pltpu.VMEM((nb, nrows, deg), jnp.float32),
            [pltpu.SemaphoreType.DMA] * nb,
            [pltpu.SemaphoreType.DMA] * nb,
            [pltpu.SemaphoreType.DMA] * nb,
        ],
        compiler_params=pltpu.CompilerParams(needs_layout_passes=False),
    )
    def gather_kernel(feat_hbm, cx_hbm, cy_hbm, cz_hbm, src_hbm,
                      gf_hbm, xx_hbm, xy_hbm, xz_hbm,
                      idx_v, fbuf, cxv, cyv, czv,
                      xb, yb, zb, gsems, ssems, csems):
        wid = lax.axis_index("s") * 2 + lax.axis_index("c")
        pltpu.sync_copy(src_hbm.at[wid], idx_v)
        base = wid * nchunk
        for b in range(nb):
            pltpu.async_copy(feat_hbm.at[idx_v.at[b]], fbuf.at[b], gsems[b])
        pltpu.sync_copy(cx_hbm, cxv)
        pltpu.sync_copy(cy_hbm, cyv)
        pltpu.sync_copy(cz_hbm, czv)

        def ring(jj, carry):
            j0 = jj * nb
            for b in range(nb):
                j = j0 + b
                pltpu.make_async_copy(
                    feat_hbm.at[idx_v.at[j]], fbuf.at[b], gsems[b]).wait()
                row = (base + j) * ch
                pltpu.async_copy(fbuf.at[b], gf_hbm.at[pl.ds(row, ch)],
                                 ssems[b])
                nrow = (base + j) * nrows

                @pl.when(j >= nb)
                def _():
                    pltpu.make_async_copy(
                        xb.at[b], xx_hbm.at[pl.ds(nrow, nrows)],
                        csems[b]).wait()
                    pltpu.make_async_copy(
                        yb.at[b], xy_hbm.at[pl.ds(nrow, nrows)],
                        csems[b]).wait()
                    pltpu.make_async_copy(
                        zb.at[b], xz_hbm.at[pl.ds(nrow, nrows)],
                        csems[b]).wait()

                for t in range(nrows):
                    iv = idx_v[j, pl.ds(t * 16, 16)]
                    xb.at[b][t, :] = plsc.load_gather(cxv, [iv])
                    yb.at[b][t, :] = plsc.load_gather(cyv, [iv])
                    zb.at[b][t, :] = plsc.load_gather(czv, [iv])
                pltpu.async_copy(xb.at[b], xx_hbm.at[pl.ds(nrow, nrows)],
                                 csems[b])
                pltpu.async_copy(yb.at[b], xy_hbm.at[pl.ds(nrow, nrows)],
                                 csems[b])
                pltpu.async_copy(zb.at[b], xz_hbm.at[pl.ds(nrow, nrows)],
                                 csems[b])
                nxt = j + nb

                @pl.when(nxt < nchunk)
                def _():
                    pltpu.make_async_copy(
                        fbuf.at[b], gf_hbm.at[pl.ds(row, ch)], ssems[b]).wait()
                    pltpu.async_copy(
                        feat_hbm.at[idx_v.at[nxt]], fbuf.at[b], gsems[b])
            return carry

        lax.fori_loop(0, nchunk // nb, ring, 0)
        for b in range(nb):
            pltpu.make_async_copy(
                fbuf.at[b], gf_hbm.at[pl.ds(base * ch, ch)], ssems[b]).wait()
            for cb0, oh in ((xb, xx_hbm), (yb, xy_hbm), (zb, xz_hbm)):
                pltpu.make_async_copy(
                    cb0.at[b], oh.at[pl.ds(base * nrows, nrows)],
                    csems[b]).wait()

    return gather_kernel(feat, cx, cy, cz, src_pad)


def _tc_project(feat, ew1b, ew1c):
    n, d = feat.shape
    h = ew1b.shape[1]
    bp = 2000
    grid = n // bp

    def kern(feat_ref, wb_ref, wc_ref, zs_ref, zd_ref):
        f = feat_ref[...]
        zs_ref[...] = jnp.dot(f, wb_ref[...], preferred_element_type=jnp.float32)
        zd_ref[...] = jnp.dot(f, wc_ref[...], preferred_element_type=jnp.float32)

    return pl.pallas_call(
        kern,
        grid=(grid,),
        in_specs=[
            pl.BlockSpec((bp, d), lambda i: (i, 0)),
            pl.BlockSpec((d, h), lambda i: (0, 0)),
            pl.BlockSpec((d, h), lambda i: (0, 0)),
        ],
        out_specs=[
            pl.BlockSpec((bp, h), lambda i: (i, 0)),
            pl.BlockSpec((bp, h), lambda i: (i, 0)),
        ],
        out_shape=[
            jax.ShapeDtypeStruct((n, h), jnp.float32),
            jax.ShapeDtypeStruct((n, h), jnp.float32),
        ],
        compiler_params=pltpu.CompilerParams(
            dimension_semantics=("parallel",)),
    )(feat, ew1b, ew1c)


def _tc_total(xx, xy, xz, deg, n):
    bp = 2000
    grid = n // bp

    def kern(xx_ref, xy_ref, xz_ref, out_ref):
        @pl.when(pl.program_id(0) == 0)
        def _():
            out_ref[...] = jnp.zeros((1, 1), jnp.float32)

        acc = jnp.float32(0.0)
        for r in (xx_ref, xy_ref, xz_ref):
            x = r[...]
            rs = jnp.sum(x, axis=1)
            acc += 2.0 * deg * jnp.sum(x * x) - 2.0 * jnp.sum(rs * rs)
        out_ref[...] += jnp.reshape(acc, (1, 1))

    return pl.pallas_call(
        kern,
        grid=(grid,),
        in_specs=[pl.BlockSpec((bp, deg), lambda i: (i, 0))] * 3,
        out_specs=pl.BlockSpec((1, 1), lambda i: (0, 0)),
        out_shape=jax.ShapeDtypeStruct((1, 1), jnp.float32),
        compiler_params=pltpu.CompilerParams(
            dimension_semantics=("arbitrary",)),
    )(xx, xy, xz)


def _tc_main(g, feat, coordinate, zdst, xx, xy, xz, ownm, total, w, bn, deg):
    n, d = feat.shape
    hs = 8
    be = bn * deg
    grid = n // bn

    def kern(g_ref, feat_ref, coord_ref, zdst_ref, xx_ref, xy_ref, xz_ref,
             ownm_ref, tot_ref,
             r_expand, w1t, b1t, bd2, b2t, ssel,
             bde_sm, bde_mx, bde_mn, bde_sd, esb128, mask8,
             nsw, nsb,
             ew1at, ew1d, eb1, ew2, eb2,
             cw1, cb1, cw2, cb2,
             nw1a, nw1b, nw1c, nb1, nw2, nb2,
             hout_ref, xout_ref):
        inv_total = 1.0 / (tot_ref[0, 0] + 1.0)

        ownm = ownm_ref[...]
        delta = jnp.zeros((be, deg), jnp.float32)
        xis = []
        for xref in (xx_ref, xy_ref, xz_ref):
            xc = xref[...]
            xc_rep = jnp.broadcast_to(
                xc[:, None, :], (bn, deg, deg)).reshape(be, deg)
            xi = jnp.sum(xc_rep * ownm, axis=1, keepdims=True)
            xis.append(xi)
            dcomp = xi - xc_rep
            delta = delta + dcomp * dcomp
        delta = delta * inv_total

        delta_rep = jnp.dot(delta, r_expand[...],
                            preferred_element_type=jnp.float32)
        h1 = _silu(delta_rep * w1t[...] + b1t[...])
        h2 = _silu(jnp.dot(h1, bd2[...],
                           preferred_element_type=jnp.float32) + b2t[...])

        h3 = h2.reshape(bn, deg, deg * hs)
        s1p = jnp.sum(h3, axis=1)
        sq1p = jnp.sum(h3 * h3, axis=1)
        mx1p = jnp.max(h3, axis=1)
        mn1p = jnp.min(h3, axis=1)
        mean1p = s1p * (1.0 / deg)
        std1p = jnp.sqrt(jnp.maximum(
            sq1p * (1.0 / deg) - mean1p * mean1p, 0.0))
        hedp = _silu(
            jnp.dot(s1p, bde_sm[...], preferred_element_type=jnp.float32)
            + jnp.dot(mx1p, bde_mx[...], preferred_element_type=jnp.float32)
            + jnp.dot(mn1p, bde_mn[...], preferred_element_type=jnp.float32)
            + jnp.dot(std1p, bde_sd[...], preferred_element_type=jnp.float32)
            + esb128[...])

        s2 = jnp.dot(hedp, ssel[...], preferred_element_type=jnp.float32)
        sq2 = jnp.dot(hedp * hedp, ssel[...], preferred_element_type=jnp.float32)
        mx2 = hedp
        mn2 = hedp
        width = deg * hs
        while width > hs:
            half = width // 2
            mx2 = jnp.maximum(mx2[:, :half], mx2[:, half:width])
            mn2 = jnp.minimum(mn2[:, :half], mn2[:, half:width])
            width = half
        mean2 = s2 * (1.0 / deg)
        std2 = jnp.sqrt(jnp.maximum(sq2 * (1.0 / deg) - mean2 * mean2, 0.0))
        pna2 = jnp.concatenate([s2, mean2, mx2, mn2, std2], axis=1)
        h_v_dx = _silu(jnp.dot(pna2, nsw[...],
                               preferred_element_type=jnp.float32) + nsb[...])

        cdst = coord_ref[...]
        xi3 = jnp.concatenate(xis, axis=1)
        cdst_rep = jnp.broadcast_to(
            cdst[:, None, :], (bn, deg, 3)).reshape(be, 3)
        dv3 = xi3 - cdst_rep
        sqd = jnp.sum(dv3 * dv3, axis=1, keepdims=True)
        fblk = feat_ref[...]
        zdst = zdst_ref[...]
        zdst_rep = jnp.broadcast_to(
            zdst[:, None, :], (bn, deg, zdst.shape[1])).reshape(be, -1)
        hedp_rep = jnp.broadcast_to(
            hedp[:, None, :], (bn, deg, deg * hs)).reshape(be, deg * hs)
        z1 = (jnp.dot(hedp_rep * mask8[...], ew1at[...],
                      preferred_element_type=jnp.float32)
              + g_ref[...] + zdst_rep + sqd * ew1d[...] + eb1[...])
        h_e = _silu(jnp.dot(_silu(z1), ew2[...],
                            preferred_element_type=jnp.float32) + eb2[...])

        t = _silu(jnp.dot(h_e, cw1[...],
                          preferred_element_type=jnp.float32) + cb1[...])
        coef = jnp.dot(t, cw2[...],
                       preferred_element_type=jnp.float32) + cb2[...]
        x_e = dv3 * coef
        x_agg = jnp.sum(x_e.reshape(bn, deg, 3), axis=1)
        xout_ref[...] = cdst + x_agg

        h_agg = jnp.sum(h_e.reshape(bn, deg, d), axis=1)
        z = (jnp.dot(fblk, nw1a[...], preferred_element_type=jnp.float32)
             + jnp.dot(h_agg, nw1b[...], preferred_element_type=jnp.float32)
             + jnp.dot(h_v_dx, nw1c[...], preferred_element_type=jnp.float32)
             + nb1[...])
        hout_ref[...] = jnp.dot(_silu(z), nw2[...],
                                preferred_element_type=jnp.float32) + nb2[...]

    const = lambda a: pl.BlockSpec(a.shape, lambda i: (0,) * a.ndim)
    weights = [w[k] for k in (
        "r_expand", "w1t", "b1t", "bd2", "b2t", "ssel",
        "bde_sm", "bde_mx", "bde_mn", "bde_sd", "esb128", "mask8",
        "nsw", "nsb",
        "ew1at", "ew1d", "eb1", "ew2", "eb2",
        "cw1", "cb1", "cw2", "cb2",
        "nw1a", "nw1b", "nw1c", "nb1", "nw2", "nb2")]
    in_specs = [
        pl.BlockSpec((be, d), lambda i: (i, 0)),
        pl.BlockSpec((bn, d), lambda i: (i, 0)),
        pl.BlockSpec((bn, 3), lambda i: (i, 0)),
        pl.BlockSpec((bn, d), lambda i: (i, 0)),
        pl.BlockSpec((bn, deg), lambda i: (i, 0)),
        pl.BlockSpec((bn, deg), lambda i: (i, 0)),
        pl.BlockSpec((bn, deg), lambda i: (i, 0)),
        pl.BlockSpec((be, deg), lambda i: (0, 0)),
        pl.BlockSpec((1, 1), lambda i: (0, 0)),
    ] + [const(a) for a in weights]
    return pl.pallas_call(
        kern,
        grid=(grid,),
        in_specs=in_specs,
        out_specs=[
            pl.BlockSpec((bn, d), lambda i: (i, 0)),
            pl.BlockSpec((bn, 3), lambda i: (i, 0)),
        ],
        out_shape=[
            jax.ShapeDtypeStruct((n, d), jnp.float32),
            jax.ShapeDtypeStruct((n, 3), jnp.float32),
        ],
        compiler_params=pltpu.CompilerParams(
            dimension_semantics=("parallel",)),
    )(g, feat, coordinate, zdst, xx, xy, xz, ownm, total, *weights)


def _prep_weights(p, d, deg, hs, be):
    h = p["eW2"].shape[0]
    jidx = jnp.arange(deg * hs) // hs
    r_expand = (jnp.arange(deg)[:, None] == jidx[None, :]).astype(jnp.float32)
    ssel = (jnp.arange(deg * hs)[:, None] % hs
            == jnp.arange(hs)[None, :]).astype(jnp.float32)
    bd2 = jnp.kron(jnp.eye(deg, dtype=jnp.float32), p["dW2"])
    eye16 = jnp.eye(deg, dtype=jnp.float32)
    esw = p["esW"]
    mask8 = ((jnp.arange(deg * hs)[None, :] // hs)
             == (jnp.arange(be)[:, None] % deg)).astype(jnp.float32)
    w = {
        "r_expand": r_expand,
        "w1t": jnp.tile(p["dW1"][0], deg)[None, :],
        "b1t": jnp.tile(p["db1"], deg)[None, :],
        "bd2": bd2,
        "b2t": jnp.tile(p["db2"], deg)[None, :],
        "ssel": ssel,
        "bde_sm": jnp.kron(eye16, esw[:hs] + esw[hs:2 * hs] / deg),
        "bde_mx": jnp.kron(eye16, esw[2 * hs:3 * hs]),
        "bde_mn": jnp.kron(eye16, esw[3 * hs:4 * hs]),
        "bde_sd": jnp.kron(eye16, esw[4 * hs:5 * hs]),
        "esb128": jnp.tile(p["esb"], deg)[None, :],
        "mask8": mask8,
        "nsw": p["nsW"],
        "nsb": p["nsb"][None, :],
        "ew1at": jnp.tile(p["eW1"][:hs], (deg, 1)),
        "ew1d": p["eW1"][hs + 2 * d:hs + 2 * d + 1],
        "eb1": p["eb1"][None, :],
        "ew2": p["eW2"],
        "eb2": p["eb2"][None, :],
        "cw1": p["cW1"],
        "cb1": p["cb1"][None, :],
        "cw2": p["cW2"],
        "cb2": p["cb2"][None, :],
        "nw1a": p["nW1"][:d],
        "nw1b": p["nW1"][d:2 * d],
        "nw1c": p["nW1"][2 * d:2 * d + hs],
        "nb1": p["nb1"][None, :],
        "nw2": p["nW2"],
        "nb2": p["nb2"][None, :],
    }
    return w


def kernel(feat, coordinate, edge_index, params):
    n, d = feat.shape
    e = edge_index.shape[1]
    deg = e // n
    hs = params["dW2"].shape[0]
    src = edge_index[0].astype(jnp.int32)

    nchunk = -(-e // (_NW * _CH))
    epad = _NW * _CH * nchunk
    src_pad = jnp.pad(src, (0, epad - e)).reshape(_NW, nchunk, _CH)

    zsrc, zdst = _tc_project(
        feat, params["eW1"][hs:hs + d], params["eW1"][hs + d:hs + 2 * d])
    g, xx, xy, xz = _sc_gather(
        zsrc, coordinate[:, 0], coordinate[:, 1], coordinate[:, 2], src_pad)

    total = _tc_total(xx, xy, xz, deg, n)
    bn = 200
    w = _prep_weights(params, d, deg, hs, bn * deg)
    ownm = (jnp.arange(bn * deg)[:, None] % deg
            == jnp.arange(deg)[None, :]).astype(jnp.float32)
    h_new, x_new = _tc_main(
        g, feat, coordinate, zdst, xx, xy, xz, ownm, total, w, bn, deg)
    return h_new, x_new


# speedup vs baseline: 1.5292x; 1.2121x over previous
import functools

import jax
import jax.numpy as jnp
from jax import lax
from jax.experimental import pallas as pl
from jax.experimental.pallas import tpu as pltpu
from jax.experimental.pallas import tpu_sc as plsc

_NW = 32
_CH = 128


def _silu(x):
    return x * jax.nn.sigmoid(x)


def _sc_gather_coords(cx, cy, cz, src_pad):
    nw, nchunk, ch = src_pad.shape
    n = cx.shape[0]
    deg = 16
    epad = nw * nchunk * ch
    npad = epad // deg
    nrows = ch // deg
    wrows = nchunk * nrows
    mesh = plsc.VectorSubcoreMesh(core_axis_name="c", subcore_axis_name="s")

    @functools.partial(
        pl.kernel,
        out_type=(
            jax.ShapeDtypeStruct((npad, deg), jnp.float32),
            jax.ShapeDtypeStruct((npad, deg), jnp.float32),
            jax.ShapeDtypeStruct((npad, deg), jnp.float32),
        ),
        mesh=mesh,
        scratch_types=[
            pltpu.VMEM((nchunk, ch), jnp.int32),
            pltpu.VMEM((n,), jnp.float32),
            pltpu.VMEM((n,), jnp.float32),
            pltpu.VMEM((n,), jnp.float32),
            pltpu.VMEM((2, nrows, deg), jnp.float32),
            pltpu.VMEM((2, nrows, deg), jnp.float32),
            pltpu.VMEM((2, nrows, deg), jnp.float32),
            [pltpu.SemaphoreType.DMA] * 2,
        ],
        compiler_params=pltpu.CompilerParams(needs_layout_passes=False),
    )
    def coord_kernel(cx_hbm, cy_hbm, cz_hbm, src_hbm,
                     xx_hbm, xy_hbm, xz_hbm,
                     idx_v, cxv, cyv, czv, xb, yb, zb, csems):
        wid = lax.axis_index("s") * 2 + lax.axis_index("c")
        pltpu.sync_copy(src_hbm.at[wid], idx_v)
        pltpu.sync_copy(cx_hbm, cxv)
        pltpu.sync_copy(cy_hbm, cyv)
        pltpu.sync_copy(cz_hbm, czv)
        base = wid * nchunk

        def cgather(jj, carry):
            for b in range(2):
                j = jj * 2 + b
                nrow = (base + j) * nrows

                @pl.when(j >= 2)
                def _():
                    for cb0, oh in ((xb, xx_hbm), (yb, xy_hbm), (zb, xz_hbm)):
                        pltpu.make_async_copy(
                            cb0.at[b], oh.at[pl.ds(nrow, nrows)],
                            csems[b]).wait()

                for t in range(nrows):
                    iv = idx_v[j, pl.ds(t * 16, 16)]
                    xb.at[b][t, :] = plsc.load_gather(cxv, [iv])
                    yb.at[b][t, :] = plsc.load_gather(cyv, [iv])
                    zb.at[b][t, :] = plsc.load_gather(czv, [iv])
                for cb0, oh in ((xb, xx_hbm), (yb, xy_hbm), (zb, xz_hbm)):
                    pltpu.async_copy(
                        cb0.at[b], oh.at[pl.ds(nrow, nrows)], csems[b])
            return carry

        lax.fori_loop(0, nchunk // 2, cgather, 0)
        for b in range(2):
            for cb0, oh in ((xb, xx_hbm), (yb, xy_hbm), (zb, xz_hbm)):
                pltpu.make_async_copy(
                    cb0.at[b], oh.at[pl.ds(base * nrows, nrows)],
                    csems[b]).wait()

    return coord_kernel(cx, cy, cz, src_pad)


def _sc_gather(feat, src_pad):
    nw, nchunk, ch = src_pad.shape
    n, d = feat.shape
    epad = nw * nchunk * ch
    mesh = plsc.VectorSubcoreMesh(core_axis_name="c", subcore_axis_name="s")

    nb = 2

    @functools.partial(
        pl.kernel,
        out_type=jax.ShapeDtypeStruct((epad, d), jnp.float32),
        mesh=mesh,
        scratch_types=[
            pltpu.VMEM((nchunk, ch), jnp.int32),
            pltpu.VMEM((nb, ch, d), jnp.float32),
            [pltpu.SemaphoreType.DMA] * nb,
            [pltpu.SemaphoreType.DMA] * nb,
        ],
        compiler_params=pltpu.CompilerParams(needs_layout_passes=False),
    )
    def gather_kernel(feat_hbm, src_hbm, gf_hbm, idx_v, fbuf, gsems, ssems):
        wid = lax.axis_index("s") * 2 + lax.axis_index("c")
        pltpu.sync_copy(src_hbm.at[wid], idx_v)
        base = wid * nchunk
        for b in range(nb):
            pltpu.async_copy(feat_hbm.at[idx_v.at[b]], fbuf.at[b], gsems[b])

        def ring(jj, carry):
            j0 = jj * nb
            for b in range(nb):
                j = j0 + b
                pltpu.make_async_copy(
                    feat_hbm.at[idx_v.at[j]], fbuf.at[b], gsems[b]).wait()
                row = (base + j) * ch
                pltpu.async_copy(fbuf.at[b], gf_hbm.at[pl.ds(row, ch)],
                                 ssems[b])
                nxt = j + nb

                @pl.when(nxt < nchunk)
                def _():
                    pltpu.make_async_copy(
                        fbuf.at[b], gf_hbm.at[pl.ds(row, ch)], ssems[b]).wait()
                    pltpu.async_copy(
                        feat_hbm.at[idx_v.at[nxt]], fbuf.at[b], gsems[b])
            return carry

        lax.fori_loop(0, nchunk // nb, ring, 0)
        for b in range(nb):
            pltpu.make_async_copy(
                fbuf.at[b], gf_hbm.at[pl.ds(base * ch, ch)], ssems[b]).wait()

    return gather_kernel(feat, src_pad)


def _tc_project(feat, ew1b, ew1c):
    n, d = feat.shape
    h = ew1b.shape[1]
    bp = 2000
    grid = n // bp

    def kern(feat_ref, wb_ref, wc_ref, zs_ref, zd_ref):
        f = feat_ref[...]
        zs_ref[...] = jnp.dot(f, wb_ref[...], preferred_element_type=jnp.float32)
        zd_ref[...] = jnp.dot(f, wc_ref[...], preferred_element_type=jnp.float32)

    return pl.pallas_call(
        kern,
        grid=(grid,),
        in_specs=[
            pl.BlockSpec((bp, d), lambda i: (i, 0)),
            pl.BlockSpec((d, h), lambda i: (0, 0)),
            pl.BlockSpec((d, h), lambda i: (0, 0)),
        ],
        out_specs=[
            pl.BlockSpec((bp, h), lambda i: (i, 0)),
            pl.BlockSpec((bp, h), lambda i: (i, 0)),
        ],
        out_shape=[
            jax.ShapeDtypeStruct((n, h), jnp.float32),
            jax.ShapeDtypeStruct((n, h), jnp.float32),
        ],
        compiler_params=pltpu.CompilerParams(
            dimension_semantics=("parallel",)),
    )(feat, ew1b, ew1c)


def _tc_total(xx, xy, xz, deg, n):
    bp = 2000
    grid = n // bp

    def kern(xx_ref, xy_ref, xz_ref, out_ref):
        @pl.when(pl.program_id(0) == 0)
        def _():
            out_ref[...] = jnp.zeros((1, 1), jnp.float32)

        acc = jnp.float32(0.0)
        for r in (xx_ref, xy_ref, xz_ref):
            x = r[...]
            rs = jnp.sum(x, axis=1)
            acc += 2.0 * deg * jnp.sum(x * x) - 2.0 * jnp.sum(rs * rs)
        out_ref[...] += jnp.reshape(acc, (1, 1))

    return pl.pallas_call(
        kern,
        grid=(grid,),
        in_specs=[pl.BlockSpec((bp, deg), lambda i: (i, 0))] * 3,
        out_specs=pl.BlockSpec((1, 1), lambda i: (0, 0)),
        out_shape=jax.ShapeDtypeStruct((1, 1), jnp.float32),
        compiler_params=pltpu.CompilerParams(
            dimension_semantics=("arbitrary",)),
    )(xx, xy, xz)


def _tc_stage_a(xx, xy, xz, ownm, total, w, bn, deg, n):
    hs = 8
    be = bn * deg
    grid = n // bn

    def kern(xx_ref, xy_ref, xz_ref, ownm_ref, tot_ref,
             r_expand, w1t, b1t, bd2, b2t, ssel,
             bde_sm, bde_mx, bde_mn, bde_sd, esb128,
             nsw, nsb,
             hedp_ref, hv_ref):
        inv_total = 1.0 / (tot_ref[0, 0] + 1.0)
        ownm = ownm_ref[...]
        delta = jnp.zeros((be, deg), jnp.float32)
        for xref in (xx_ref, xy_ref, xz_ref):
            xc = xref[...]
            xc_rep = jnp.broadcast_to(
                xc[:, None, :], (bn, deg, deg)).reshape(be, deg)
            xi = jnp.sum(xc_rep * ownm, axis=1, keepdims=True)
            dcomp = xi - xc_rep
            delta = delta + dcomp * dcomp
        delta = delta * inv_total

        delta_rep = jnp.dot(delta, r_expand[...],
                            preferred_element_type=jnp.float32)
        h1 = _silu(delta_rep * w1t[...] + b1t[...])
        h2 = _silu(jnp.dot(h1, bd2[...],
                           preferred_element_type=jnp.float32) + b2t[...])

        h3 = h2.reshape(bn, deg, deg * hs)
        s1p = jnp.sum(h3, axis=1)
        sq1p = jnp.sum(h3 * h3, axis=1)
        mx1p = jnp.max(h3, axis=1)
        mn1p = jnp.min(h3, axis=1)
        mean1p = s1p * (1.0 / deg)
        std1p = jnp.sqrt(jnp.maximum(
            sq1p * (1.0 / deg) - mean1p * mean1p, 0.0))
        hedp = _silu(
            jnp.dot(s1p, bde_sm[...], preferred_element_type=jnp.float32)
            + jnp.dot(mx1p, bde_mx[...], preferred_element_type=jnp.float32)
            + jnp.dot(mn1p, bde_mn[...], preferred_element_type=jnp.float32)
            + jnp.dot(std1p, bde_sd[...], preferred_element_type=jnp.float32)
            + esb128[...])

        s2 = jnp.dot(hedp, ssel[...], preferred_element_type=jnp.float32)
        sq2 = jnp.dot(hedp * hedp, ssel[...], preferred_element_type=jnp.float32)
        mx2 = hedp
        mn2 = hedp
        width = deg * hs
        while width > hs:
            half = width // 2
            mx2 = jnp.maximum(mx2[:, :half], mx2[:, half:width])
            mn2 = jnp.minimum(mn2[:, :half], mn2[:, half:width])
            width = half
        mean2 = s2 * (1.0 / deg)
        std2 = jnp.sqrt(jnp.maximum(sq2 * (1.0 / deg) - mean2 * mean2, 0.0))
        pna2 = jnp.concatenate([s2, mean2, mx2, mn2, std2], axis=1)
        hv_ref[...] = _silu(jnp.dot(pna2, nsw[...],
                                    preferred_element_type=jnp.float32)
                            + nsb[...])
        hedp_ref[...] = hedp

    const = lambda a: pl.BlockSpec(a.shape, lambda i: (0,) * a.ndim)
    weights = [w[k] for k in (
        "r_expand", "w1t", "b1t", "bd2", "b2t", "ssel",
        "bde_sm", "bde_mx", "bde_mn", "bde_sd", "esb128",
        "nsw", "nsb")]
    in_specs = [
        pl.BlockSpec((bn, deg), lambda i: (i, 0)),
        pl.BlockSpec((bn, deg), lambda i: (i, 0)),
        pl.BlockSpec((bn, deg), lambda i: (i, 0)),
        pl.BlockSpec((be, deg), lambda i: (0, 0)),
        pl.BlockSpec((1, 1), lambda i: (0, 0)),
    ] + [const(a) for a in weights]
    return pl.pallas_call(
        kern,
        grid=(grid,),
        in_specs=in_specs,
        out_specs=[
            pl.BlockSpec((bn, deg * hs), lambda i: (i, 0)),
            pl.BlockSpec((bn, hs), lambda i: (i, 0)),
        ],
        out_shape=[
            jax.ShapeDtypeStruct((n, deg * hs), jnp.float32),
            jax.ShapeDtypeStruct((n, hs), jnp.float32),
        ],
        compiler_params=pltpu.CompilerParams(
            dimension_semantics=("parallel",)),
    )(xx, xy, xz, ownm, total, *weights)


def _tc_stage_b(g, feat, coordinate, zdst, xx, xy, xz, ownm, hedp_a, hv_a,
                w, bn, deg):
    n, d = feat.shape
    hs = 8
    be = bn * deg
    grid = n // bn

    def kern(g_ref, feat_ref, coord_ref, zdst_ref, xx_ref, xy_ref, xz_ref,
             ownm_ref, hedp_ref, hv_ref,
             mask8, ew1at, ew1d, eb1, ew2, eb2,
             cw1, cb1, cw2, cb2,
             nw1a, nw1b, nw1c, nb1, nw2, nb2,
             hout_ref, xout_ref):
        ownm = ownm_ref[...]
        xis = []
        for xref in (xx_ref, xy_ref, xz_ref):
            xc = xref[...]
            xc_rep = jnp.broadcast_to(
                xc[:, None, :], (bn, deg, deg)).reshape(be, deg)
            xis.append(jnp.sum(xc_rep * ownm, axis=1, keepdims=True))
        hedp = hedp_ref[...]
        h_v_dx = hv_ref[...]

        cdst = coord_ref[...]
        xi3 = jnp.concatenate(xis, axis=1)
        cdst_rep = jnp.broadcast_to(
            cdst[:, None, :], (bn, deg, 3)).reshape(be, 3)
        dv3 = xi3 - cdst_rep
        sqd = jnp.sum(dv3 * dv3, axis=1, keepdims=True)
        fblk = feat_ref[...]
        zdst = zdst_ref[...]
        zdst_rep = jnp.broadcast_to(
            zdst[:, None, :], (bn, deg, zdst.shape[1])).reshape(be, -1)
        hedp_rep = jnp.broadcast_to(
            hedp[:, None, :], (bn, deg, deg * hs)).reshape(be, deg * hs)
        z1 = (jnp.dot(hedp_rep * mask8[...], ew1at[...],
                      preferred_element_type=jnp.float32)
              + g_ref[...] + zdst_rep + sqd * ew1d[...] + eb1[...])
        h_e = _silu(jnp.dot(_silu(z1), ew2[...],
                            preferred_element_type=jnp.float32) + eb2[...])

        t = _silu(jnp.dot(h_e, cw1[...],
                          preferred_element_type=jnp.float32) + cb1[...])
        coef = jnp.dot(t, cw2[...],
                       preferred_element_type=jnp.float32) + cb2[...]
        x_e = dv3 * coef
        x_agg = jnp.sum(x_e.reshape(bn, deg, 3), axis=1)
        xout_ref[...] = cdst + x_agg

        h_agg = jnp.sum(h_e.reshape(bn, deg, d), axis=1)
        z = (jnp.dot(fblk, nw1a[...], preferred_element_type=jnp.float32)
             + jnp.dot(h_agg, nw1b[...], preferred_element_type=jnp.float32)
             + jnp.dot(h_v_dx, nw1c[...], preferred_element_type=jnp.float32)
             + nb1[...])
        hout_ref[...] = jnp.dot(_silu(z), nw2[...],
                                preferred_element_type=jnp.float32) + nb2[...]

    const = lambda a: pl.BlockSpec(a.shape, lambda i: (0,) * a.ndim)
    weights = [w[k] for k in (
        "mask8",
        "ew1at", "ew1d", "eb1", "ew2", "eb2",
        "cw1", "cb1", "cw2", "cb2",
        "nw1a", "nw1b", "nw1c", "nb1", "nw2", "nb2")]
    in_specs = [
        pl.BlockSpec((be, d), lambda i: (i, 0)),
        pl.BlockSpec((bn, d), lambda i: (i, 0)),
        pl.BlockSpec((bn, 3), lambda i: (i, 0)),
        pl.BlockSpec((bn, d), lambda i: (i, 0)),
        pl.BlockSpec((bn, deg), lambda i: (i, 0)),
        pl.BlockSpec((bn, deg), lambda i: (i, 0)),
        pl.BlockSpec((bn, deg), lambda i: (i, 0)),
        pl.BlockSpec((be, deg), lambda i: (0, 0)),
        pl.BlockSpec((bn, deg * hs), lambda i: (i, 0)),
        pl.BlockSpec((bn, hs), lambda i: (i, 0)),
    ] + [const(a) for a in weights]
    return pl.pallas_call(
        kern,
        grid=(grid,),
        in_specs=in_specs,
        out_specs=[
            pl.BlockSpec((bn, d), lambda i: (i, 0)),
            pl.BlockSpec((bn, 3), lambda i: (i, 0)),
        ],
        out_shape=[
            jax.ShapeDtypeStruct((n, d), jnp.float32),
            jax.ShapeDtypeStruct((n, 3), jnp.float32),
        ],
        compiler_params=pltpu.CompilerParams(
            dimension_semantics=("parallel",)),
    )(g, feat, coordinate, zdst, xx, xy, xz, ownm, hedp_a, hv_a, *weights)


def _prep_weights(p, d, deg, hs, be):
    h = p["eW2"].shape[0]
    jidx = jnp.arange(deg * hs) // hs
    r_expand = (jnp.arange(deg)[:, None] == jidx[None, :]).astype(jnp.float32)
    ssel = (jnp.arange(deg * hs)[:, None] % hs
            == jnp.arange(hs)[None, :]).astype(jnp.float32)
    bd2 = jnp.kron(jnp.eye(deg, dtype=jnp.float32), p["dW2"])
    eye16 = jnp.eye(deg, dtype=jnp.float32)
    esw = p["esW"]
    mask8 = ((jnp.arange(deg * hs)[None, :] // hs)
             == (jnp.arange(be)[:, None] % deg)).astype(jnp.float32)
    w = {
        "r_expand": r_expand,
        "w1t": jnp.tile(p["dW1"][0], deg)[None, :],
        "b1t": jnp.tile(p["db1"], deg)[None, :],
        "bd2": bd2,
        "b2t": jnp.tile(p["db2"], deg)[None, :],
        "ssel": ssel,
        "bde_sm": jnp.kron(eye16, esw[:hs] + esw[hs:2 * hs] / deg),
        "bde_mx": jnp.kron(eye16, esw[2 * hs:3 * hs]),
        "bde_mn": jnp.kron(eye16, esw[3 * hs:4 * hs]),
        "bde_sd": jnp.kron(eye16, esw[4 * hs:5 * hs]),
        "esb128": jnp.tile(p["esb"], deg)[None, :],
        "mask8": mask8,
        "nsw": p["nsW"],
        "nsb": p["nsb"][None, :],
        "ew1at": jnp.tile(p["eW1"][:hs], (deg, 1)),
        "ew1d": p["eW1"][hs + 2 * d:hs + 2 * d + 1],
        "eb1": p["eb1"][None, :],
        "ew2": p["eW2"],
        "eb2": p["eb2"][None, :],
        "cw1": p["cW1"],
        "cb1": p["cb1"][None, :],
        "cw2": p["cW2"],
        "cb2": p["cb2"][None, :],
        "nw1a": p["nW1"][:d],
        "nw1b": p["nW1"][d:2 * d],
        "nw1c": p["nW1"][2 * d:2 * d + hs],
        "nb1": p["nb1"][None, :],
        "nw2": p["nW2"],
        "nb2": p["nb2"][None, :],
    }
    return w


def kernel(feat, coordinate, edge_index, params):
    n, d = feat.shape
    e = edge_index.shape[1]
    deg = e // n
    hs = params["dW2"].shape[0]
    src = edge_index[0].astype(jnp.int32)

    nchunk = -(-e // (_NW * _CH))
    epad = _NW * _CH * nchunk
    src_pad = jnp.pad(src, (0, epad - e)).reshape(_NW, nchunk, _CH)

    zsrc, zdst = _tc_project(
        feat, params["eW1"][hs:hs + d], params["eW1"][hs + d:hs + 2 * d])
    xx, xy, xz = _sc_gather_coords(
        coordinate[:, 0], coordinate[:, 1], coordinate[:, 2], src_pad)
    g = _sc_gather(zsrc, src_pad)

    total = _tc_total(xx, xy, xz, deg, n)
    bn = 200
    w = _prep_weights(params, d, deg, hs, bn * deg)
    ownm = (jnp.arange(bn * deg)[:, None] % deg
            == jnp.arange(deg)[None, :]).astype(jnp.float32)
    hedp_a, hv_a = _tc_stage_a(xx, xy, xz, ownm, total, w, bn, deg, n)
    h_new, x_new = _tc_stage_b(
        g, feat, coordinate, zdst, xx, xy, xz, ownm, hedp_a, hv_a, w, bn, deg)
    return h_new, x_new


# speedup vs baseline: 1.5360x; 1.0044x over previous
import functools

import jax
import jax.numpy as jnp
from jax import lax
from jax.experimental import pallas as pl
from jax.experimental.pallas import tpu as pltpu
from jax.experimental.pallas import tpu_sc as plsc

_NW = 32
_CH = 128


def _silu(x):
    return x * jax.nn.sigmoid(x)


def _sc_gather_coords(cx, cy, cz, src_pad):
    nw, nchunk, ch = src_pad.shape
    n = cx.shape[0]
    deg = 16
    epad = nw * nchunk * ch
    npad = epad // deg
    nrows = ch // deg
    wrows = nchunk * nrows
    mesh = plsc.VectorSubcoreMesh(core_axis_name="c", subcore_axis_name="s")

    @functools.partial(
        pl.kernel,
        out_type=(
            jax.ShapeDtypeStruct((npad, deg), jnp.float32),
            jax.ShapeDtypeStruct((npad, deg), jnp.float32),
            jax.ShapeDtypeStruct((npad, deg), jnp.float32),
        ),
        mesh=mesh,
        scratch_types=[
            pltpu.VMEM((nchunk, ch), jnp.int32),
            pltpu.VMEM((n,), jnp.float32),
            pltpu.VMEM((n,), jnp.float32),
            pltpu.VMEM((n,), jnp.float32),
            pltpu.VMEM((2, nrows, deg), jnp.float32),
            pltpu.VMEM((2, nrows, deg), jnp.float32),
            pltpu.VMEM((2, nrows, deg), jnp.float32),
            [pltpu.SemaphoreType.DMA] * 2,
        ],
        compiler_params=pltpu.CompilerParams(needs_layout_passes=False),
    )
    def coord_kernel(cx_hbm, cy_hbm, cz_hbm, src_hbm,
                     xx_hbm, xy_hbm, xz_hbm,
                     idx_v, cxv, cyv, czv, xb, yb, zb, csems):
        wid = lax.axis_index("s") * 2 + lax.axis_index("c")
        pltpu.sync_copy(src_hbm.at[wid], idx_v)
        pltpu.sync_copy(cx_hbm, cxv)
        pltpu.sync_copy(cy_hbm, cyv)
        pltpu.sync_copy(cz_hbm, czv)
        base = wid * nchunk

        def cgather(jj, carry):
            for b in range(2):
                j = jj * 2 + b
                nrow = (base + j) * nrows

                @pl.when(j >= 2)
                def _():
                    for cb0, oh in ((xb, xx_hbm), (yb, xy_hbm), (zb, xz_hbm)):
                        pltpu.make_async_copy(
                            cb0.at[b], oh.at[pl.ds(nrow, nrows)],
                            csems[b]).wait()

                for t in range(nrows):
                    iv = idx_v[j, pl.ds(t * 16, 16)]
                    xb.at[b][t, :] = plsc.load_gather(cxv, [iv])
                    yb.at[b][t, :] = plsc.load_gather(cyv, [iv])
                    zb.at[b][t, :] = plsc.load_gather(czv, [iv])
                for cb0, oh in ((xb, xx_hbm), (yb, xy_hbm), (zb, xz_hbm)):
                    pltpu.async_copy(
                        cb0.at[b], oh.at[pl.ds(nrow, nrows)], csems[b])
            return carry

        lax.fori_loop(0, nchunk // 2, cgather, 0)
        for b in range(2):
            for cb0, oh in ((xb, xx_hbm), (yb, xy_hbm), (zb, xz_hbm)):
                pltpu.make_async_copy(
                    cb0.at[b], oh.at[pl.ds(base * nrows, nrows)],
                    csems[b]).wait()

    return coord_kernel(cx, cy, cz, src_pad)


def _sc_gather(feat, src_pad):
    nw, nchunk, ch = src_pad.shape
    n, d = feat.shape
    epad = nw * nchunk * ch
    mesh = plsc.VectorSubcoreMesh(core_axis_name="c", subcore_axis_name="s")

    nb = 4

    @functools.partial(
        pl.kernel,
        out_type=jax.ShapeDtypeStruct((epad, d), jnp.float32),
        mesh=mesh,
        scratch_types=[
            pltpu.VMEM((nchunk, ch), jnp.int32),
            pltpu.VMEM((nb, ch, d), jnp.float32),
            [pltpu.SemaphoreType.DMA] * nb,
            [pltpu.SemaphoreType.DMA] * nb,
        ],
        compiler_params=pltpu.CompilerParams(needs_layout_passes=False),
    )
    def gather_kernel(feat_hbm, src_hbm, gf_hbm, idx_v, fbuf, gsems, ssems):
        wid = lax.axis_index("s") * 2 + lax.axis_index("c")
        pltpu.sync_copy(src_hbm.at[wid], idx_v)
        base = wid * nchunk
        for b in range(nb):
            pltpu.async_copy(feat_hbm.at[idx_v.at[b]], fbuf.at[b], gsems[b])

        def ring(jj, carry):
            j0 = jj * nb
            for b in range(nb):
                j = j0 + b
                pltpu.make_async_copy(
                    feat_hbm.at[idx_v.at[j]], fbuf.at[b], gsems[b]).wait()
                row = (base + j) * ch
                pltpu.async_copy(fbuf.at[b], gf_hbm.at[pl.ds(row, ch)],
                                 ssems[b])
                nxt = j + nb

                @pl.when(nxt < nchunk)
                def _():
                    pltpu.make_async_copy(
                        fbuf.at[b], gf_hbm.at[pl.ds(row, ch)], ssems[b]).wait()
                    pltpu.async_copy(
                        feat_hbm.at[idx_v.at[nxt]], fbuf.at[b], gsems[b])
            return carry

        lax.fori_loop(0, nchunk // nb, ring, 0)
        for b in range(nb):
            pltpu.make_async_copy(
                fbuf.at[b], gf_hbm.at[pl.ds(base * ch, ch)], ssems[b]).wait()

    return gather_kernel(feat, src_pad)


def _tc_project(feat, ew1b, ew1c):
    n, d = feat.shape
    h = ew1b.shape[1]
    bp = 2000
    grid = n // bp

    def kern(feat_ref, wb_ref, wc_ref, zs_ref, zd_ref):
        f = feat_ref[...]
        zs_ref[...] = jnp.dot(f, wb_ref[...], preferred_element_type=jnp.float32)
        zd_ref[...] = jnp.dot(f, wc_ref[...], preferred_element_type=jnp.float32)

    return pl.pallas_call(
        kern,
        grid=(grid,),
        in_specs=[
            pl.BlockSpec((bp, d), lambda i: (i, 0)),
            pl.BlockSpec((d, h), lambda i: (0, 0)),
            pl.BlockSpec((d, h), lambda i: (0, 0)),
        ],
        out_specs=[
            pl.BlockSpec((bp, h), lambda i: (i, 0)),
            pl.BlockSpec((bp, h), lambda i: (i, 0)),
        ],
        out_shape=[
            jax.ShapeDtypeStruct((n, h), jnp.float32),
            jax.ShapeDtypeStruct((n, h), jnp.float32),
        ],
        compiler_params=pltpu.CompilerParams(
            dimension_semantics=("parallel",)),
    )(feat, ew1b, ew1c)


def _tc_total(xx, xy, xz, deg, n):
    bp = 2000
    grid = n // bp

    def kern(xx_ref, xy_ref, xz_ref, out_ref):
        @pl.when(pl.program_id(0) == 0)
        def _():
            out_ref[...] = jnp.zeros((1, 1), jnp.float32)

        acc = jnp.float32(0.0)
        for r in (xx_ref, xy_ref, xz_ref):
            x = r[...]
            rs = jnp.sum(x, axis=1)
            acc += 2.0 * deg * jnp.sum(x * x) - 2.0 * jnp.sum(rs * rs)
        out_ref[...] += jnp.reshape(acc, (1, 1))

    return pl.pallas_call(
        kern,
        grid=(grid,),
        in_specs=[pl.BlockSpec((bp, deg), lambda i: (i, 0))] * 3,
        out_specs=pl.BlockSpec((1, 1), lambda i: (0, 0)),
        out_shape=jax.ShapeDtypeStruct((1, 1), jnp.float32),
        compiler_params=pltpu.CompilerParams(
            dimension_semantics=("arbitrary",)),
    )(xx, xy, xz)


def _tc_stage_a(xx, xy, xz, ownm, total, w, bn, deg, n):
    hs = 8
    be = bn * deg
    grid = n // bn

    def kern(xx_ref, xy_ref, xz_ref, ownm_ref, tot_ref,
             r_expand, w1t, b1t, bd2, b2t, ssel,
             bde_sm, bde_mx, bde_mn, bde_sd, esb128,
             nsw, nsb,
             hedp_ref, hv_ref):
        inv_total = 1.0 / (tot_ref[0, 0] + 1.0)
        ownm = ownm_ref[...]
        delta = jnp.zeros((be, deg), jnp.float32)
        for xref in (xx_ref, xy_ref, xz_ref):
            xc = xref[...]
            xc_rep = jnp.broadcast_to(
                xc[:, None, :], (bn, deg, deg)).reshape(be, deg)
            xi = jnp.sum(xc_rep * ownm, axis=1, keepdims=True)
            dcomp = xi - xc_rep
            delta = delta + dcomp * dcomp
        delta = delta * inv_total

        delta_rep = jnp.dot(delta, r_expand[...],
                            preferred_element_type=jnp.float32)
        h1 = _silu(delta_rep * w1t[...] + b1t[...])
        h2 = _silu(jnp.dot(h1, bd2[...],
                           preferred_element_type=jnp.float32) + b2t[...])

        h3 = h2.reshape(bn, deg, deg * hs)
        s1p = jnp.sum(h3, axis=1)
        sq1p = jnp.sum(h3 * h3, axis=1)
        mx1p = jnp.max(h3, axis=1)
        mn1p = jnp.min(h3, axis=1)
        mean1p = s1p * (1.0 / deg)
        std1p = jnp.sqrt(jnp.maximum(
            sq1p * (1.0 / deg) - mean1p * mean1p, 0.0))
        hedp = _silu(
            jnp.dot(s1p, bde_sm[...], preferred_element_type=jnp.float32)
            + jnp.dot(mx1p, bde_mx[...], preferred_element_type=jnp.float32)
            + jnp.dot(mn1p, bde_mn[...], preferred_element_type=jnp.float32)
            + jnp.dot(std1p, bde_sd[...], preferred_element_type=jnp.float32)
            + esb128[...])

        s2 = jnp.dot(hedp, ssel[...], preferred_element_type=jnp.float32)
        sq2 = jnp.dot(hedp * hedp, ssel[...], preferred_element_type=jnp.float32)
        mx2 = hedp
        mn2 = hedp
        width = deg * hs
        while width > hs:
            half = width // 2
            mx2 = jnp.maximum(mx2[:, :half], mx2[:, half:width])
            mn2 = jnp.minimum(mn2[:, :half], mn2[:, half:width])
            width = half
        mean2 = s2 * (1.0 / deg)
        std2 = jnp.sqrt(jnp.maximum(sq2 * (1.0 / deg) - mean2 * mean2, 0.0))
        pna2 = jnp.concatenate([s2, mean2, mx2, mn2, std2], axis=1)
        hv_ref[...] = _silu(jnp.dot(pna2, nsw[...],
                                    preferred_element_type=jnp.float32)
                            + nsb[...])
        hedp_ref[...] = hedp

    const = lambda a: pl.BlockSpec(a.shape, lambda i: (0,) * a.ndim)
    weights = [w[k] for k in (
        "r_expand", "w1t", "b1t", "bd2", "b2t", "ssel",
        "bde_sm", "bde_mx", "bde_mn", "bde_sd", "esb128",
        "nsw", "nsb")]
    in_specs = [
        pl.BlockSpec((bn, deg), lambda i: (i, 0)),
        pl.BlockSpec((bn, deg), lambda i: (i, 0)),
        pl.BlockSpec((bn, deg), lambda i: (i, 0)),
        pl.BlockSpec((be, deg), lambda i: (0, 0)),
        pl.BlockSpec((1, 1), lambda i: (0, 0)),
    ] + [const(a) for a in weights]
    return pl.pallas_call(
        kern,
        grid=(grid,),
        in_specs=in_specs,
        out_specs=[
            pl.BlockSpec((bn, deg * hs), lambda i: (i, 0)),
            pl.BlockSpec((bn, hs), lambda i: (i, 0)),
        ],
        out_shape=[
            jax.ShapeDtypeStruct((n, deg * hs), jnp.float32),
            jax.ShapeDtypeStruct((n, hs), jnp.float32),
        ],
        compiler_params=pltpu.CompilerParams(
            dimension_semantics=("parallel",)),
    )(xx, xy, xz, ownm, total, *weights)


def _tc_stage_b(g, feat, coordinate, zdst, xx, xy, xz, ownm, hedp_a, hv_a,
                w, bn, deg):
    n, d = feat.shape
    hs = 8
    be = bn * deg
    grid = n // bn

    def kern(g_ref, feat_ref, coord_ref, zdst_ref, xx_ref, xy_ref, xz_ref,
             ownm_ref, hedp_ref, hv_ref,
             mask8, ew1at, ew1d, eb1, ew2, eb2,
             cw1, cb1, cw2, cb2,
             nw1a, nw1b, nw1c, nb1, nw2, nb2,
             hout_ref, xout_ref):
        ownm = ownm_ref[...]
        xis = []
        for xref in (xx_ref, xy_ref, xz_ref):
            xc = xref[...]
            xc_rep = jnp.broadcast_to(
                xc[:, None, :], (bn, deg, deg)).reshape(be, deg)
            xis.append(jnp.sum(xc_rep * ownm, axis=1, keepdims=True))
        hedp = hedp_ref[...]
        h_v_dx = hv_ref[...]

        cdst = coord_ref[...]
        xi3 = jnp.concatenate(xis, axis=1)
        cdst_rep = jnp.broadcast_to(
            cdst[:, None, :], (bn, deg, 3)).reshape(be, 3)
        dv3 = xi3 - cdst_rep
        sqd = jnp.sum(dv3 * dv3, axis=1, keepdims=True)
        fblk = feat_ref[...]
        zdst = zdst_ref[...]
        zdst_rep = jnp.broadcast_to(
            zdst[:, None, :], (bn, deg, zdst.shape[1])).reshape(be, -1)
        hedp_rep = jnp.broadcast_to(
            hedp[:, None, :], (bn, deg, deg * hs)).reshape(be, deg * hs)
        z1 = (jnp.dot(hedp_rep * mask8[...], ew1at[...],
                      preferred_element_type=jnp.float32)
              + g_ref[...] + zdst_rep + sqd * ew1d[...] + eb1[...])
        h_e = _silu(jnp.dot(_silu(z1), ew2[...],
                            preferred_element_type=jnp.float32) + eb2[...])

        t = _silu(jnp.dot(h_e, cw1[...],
                          preferred_element_type=jnp.float32) + cb1[...])
        coef = jnp.dot(t, cw2[...],
                       preferred_element_type=jnp.float32) + cb2[...]
        x_e = dv3 * coef
        x_agg = jnp.sum(x_e.reshape(bn, deg, 3), axis=1)
        xout_ref[...] = cdst + x_agg

        h_agg = jnp.sum(h_e.reshape(bn, deg, d), axis=1)
        z = (jnp.dot(fblk, nw1a[...], preferred_element_type=jnp.float32)
             + jnp.dot(h_agg, nw1b[...], preferred_element_type=jnp.float32)
             + jnp.dot(h_v_dx, nw1c[...], preferred_element_type=jnp.float32)
             + nb1[...])
        hout_ref[...] = jnp.dot(_silu(z), nw2[...],
                                preferred_element_type=jnp.float32) + nb2[...]

    const = lambda a: pl.BlockSpec(a.shape, lambda i: (0,) * a.ndim)
    weights = [w[k] for k in (
        "mask8",
        "ew1at", "ew1d", "eb1", "ew2", "eb2",
        "cw1", "cb1", "cw2", "cb2",
        "nw1a", "nw1b", "nw1c", "nb1", "nw2", "nb2")]
    in_specs = [
        pl.BlockSpec((be, d), lambda i: (i, 0)),
        pl.BlockSpec((bn, d), lambda i: (i, 0)),
        pl.BlockSpec((bn, 3), lambda i: (i, 0)),
        pl.BlockSpec((bn, d), lambda i: (i, 0)),
        pl.BlockSpec((bn, deg), lambda i: (i, 0)),
        pl.BlockSpec((bn, deg), lambda i: (i, 0)),
        pl.BlockSpec((bn, deg), lambda i: (i, 0)),
        pl.BlockSpec((be, deg), lambda i: (0, 0)),
        pl.BlockSpec((bn, deg * hs), lambda i: (i, 0)),
        pl.BlockSpec((bn, hs), lambda i: (i, 0)),
    ] + [const(a) for a in weights]
    return pl.pallas_call(
        kern,
        grid=(grid,),
        in_specs=in_specs,
        out_specs=[
            pl.BlockSpec((bn, d), lambda i: (i, 0)),
            pl.BlockSpec((bn, 3), lambda i: (i, 0)),
        ],
        out_shape=[
            jax.ShapeDtypeStruct((n, d), jnp.float32),
            jax.ShapeDtypeStruct((n, 3), jnp.float32),
        ],
        compiler_params=pltpu.CompilerParams(
            dimension_semantics=("parallel",)),
    )(g, feat, coordinate, zdst, xx, xy, xz, ownm, hedp_a, hv_a, *weights)


def _prep_weights(p, d, deg, hs, be):
    h = p["eW2"].shape[0]
    jidx = jnp.arange(deg * hs) // hs
    r_expand = (jnp.arange(deg)[:, None] == jidx[None, :]).astype(jnp.float32)
    ssel = (jnp.arange(deg * hs)[:, None] % hs
            == jnp.arange(hs)[None, :]).astype(jnp.float32)
    bd2 = jnp.kron(jnp.eye(deg, dtype=jnp.float32), p["dW2"])
    eye16 = jnp.eye(deg, dtype=jnp.float32)
    esw = p["esW"]
    mask8 = ((jnp.arange(deg * hs)[None, :] // hs)
             == (jnp.arange(be)[:, None] % deg)).astype(jnp.float32)
    w = {
        "r_expand": r_expand,
        "w1t": jnp.tile(p["dW1"][0], deg)[None, :],
        "b1t": jnp.tile(p["db1"], deg)[None, :],
        "bd2": bd2,
        "b2t": jnp.tile(p["db2"], deg)[None, :],
        "ssel": ssel,
        "bde_sm": jnp.kron(eye16, esw[:hs] + esw[hs:2 * hs] / deg),
        "bde_mx": jnp.kron(eye16, esw[2 * hs:3 * hs]),
        "bde_mn": jnp.kron(eye16, esw[3 * hs:4 * hs]),
        "bde_sd": jnp.kron(eye16, esw[4 * hs:5 * hs]),
        "esb128": jnp.tile(p["esb"], deg)[None, :],
        "mask8": mask8,
        "nsw": p["nsW"],
        "nsb": p["nsb"][None, :],
        "ew1at": jnp.tile(p["eW1"][:hs], (deg, 1)),
        "ew1d": p["eW1"][hs + 2 * d:hs + 2 * d + 1],
        "eb1": p["eb1"][None, :],
        "ew2": p["eW2"],
        "eb2": p["eb2"][None, :],
        "cw1": p["cW1"],
        "cb1": p["cb1"][None, :],
        "cw2": p["cW2"],
        "cb2": p["cb2"][None, :],
        "nw1a": p["nW1"][:d],
        "nw1b": p["nW1"][d:2 * d],
        "nw1c": p["nW1"][2 * d:2 * d + hs],
        "nb1": p["nb1"][None, :],
        "nw2": p["nW2"],
        "nb2": p["nb2"][None, :],
    }
    return w


def kernel(feat, coordinate, edge_index, params):
    n, d = feat.shape
    e = edge_index.shape[1]
    deg = e // n
    hs = params["dW2"].shape[0]
    src = edge_index[0].astype(jnp.int32)

    nchunk = -(-e // (_NW * _CH))
    epad = _NW * _CH * nchunk
    src_pad = jnp.pad(src, (0, epad - e)).reshape(_NW, nchunk, _CH)

    zsrc, zdst = _tc_project(
        feat, params["eW1"][hs:hs + d], params["eW1"][hs + d:hs + 2 * d])
    xx, xy, xz = _sc_gather_coords(
        coordinate[:, 0], coordinate[:, 1], coordinate[:, 2], src_pad)
    g = _sc_gather(zsrc, src_pad)

    total = _tc_total(xx, xy, xz, deg, n)
    bn = 200
    w = _prep_weights(params, d, deg, hs, bn * deg)
    ownm = (jnp.arange(bn * deg)[:, None] % deg
            == jnp.arange(deg)[None, :]).astype(jnp.float32)
    hedp_a, hv_a = _tc_stage_a(xx, xy, xz, ownm, total, w, bn, deg, n)
    h_new, x_new = _tc_stage_b(
        g, feat, coordinate, zdst, xx, xy, xz, ownm, hedp_a, hv_a, w, bn, deg)
    return h_new, x_new
